# baseline (speedup 1.0000x reference)
import functools

import jax
import jax.numpy as jnp
from jax import lax
from jax.experimental import pallas as pl
from jax.experimental.pallas import tpu as pltpu
from jax.experimental.pallas import tpu_sc as plsc

N_U = 10000
N_I = 10000
D = 128
E_R = 320000
E_T = 320000
MEAN_RATE = 3.5

N_PAD = 10240
NSUB = 16
NCORE = 2
NW = NCORE * NSUB
RPT = N_PAD // NSUB
CHUNK = 128
PAD_SRC = N_U
PAD_DST = 10200
ROW_BLK = 512
GRID = N_PAD // ROW_BLK
C0_SHARE = 0.22


def _leaky(x):
    return jnp.maximum(x, 0.01 * x)


def _mesh():
    return plsc.VectorSubcoreMesh(core_axis_name="c", subcore_axis_name="s")


_GDN = lax.GatherDimensionNumbers(
    offset_dims=(), collapsed_slice_dims=(0,), start_index_map=(0,))


def _splat_lane(vec16, j):
    idx = jnp.full((16, 1), j, jnp.int32)
    return lax.gather(vec16, idx, _GDN, slice_sizes=(1,),
                      mode=lax.GatherScatterMode.PROMISE_IN_BOUNDS)


def _shuffle(vec16, idx):
    return lax.gather(vec16, idx[:, None], _GDN, slice_sizes=(1,),
                      mode=lax.GatherScatterMode.PROMISE_IN_BOUNDS)


def _hsum16(x):
    lanes = jnp.arange(16, dtype=jnp.int32)
    for off in (8, 4, 2, 1):
        x = x + _shuffle(x, lanes ^ off)
    return x


def _fill_vec(ref, n, val):
    v = jnp.full((16,), val, jnp.float32)

    def zb(i, c):
        ref[pl.ds(i * 16, 16)] = v
        return c

    lax.fori_loop(0, n // 16, zb, None)


def _zero_rows(ref):
    z = jnp.zeros((16,), jnp.float32)

    def zb(r, c):
        for col in range(D // 16):
            ref[r, pl.ds(col * 16, 16)] = z
        return c

    lax.fori_loop(0, CHUNK, zb, None)


def _pad_edges(idx, e_pad, fill):
    return jnp.concatenate(
        [idx.astype(jnp.int32), jnp.full((e_pad - idx.shape[0],), fill, jnp.int32)])


def _sc_degrees(idx4):
    nch = idx4.shape[1] // NW

    @functools.partial(
        pl.kernel, mesh=_mesh(),
        out_type=jax.ShapeDtypeStruct((NCORE, 4, N_PAD), jnp.float32),
        scratch_types=[
            [pltpu.VMEM((nch, CHUNK), jnp.int32) for _ in range(4)],
            pltpu.VMEM((CHUNK,), jnp.float32),
            [pltpu.VMEM_SHARED((N_PAD,), jnp.float32) for _ in range(4)],
            [pltpu.SemaphoreType.DMA for _ in range(4)],
        ],
    )
    def k(idx_hbm, out_hbm, idxs, vbuf, accs, sems):
        cid = lax.axis_index("c")
        sid = lax.axis_index("s")
        wid = cid * NSUB + sid
        _fill_vec(vbuf, CHUNK, 0.0)
        for a in accs:
            for b in range(RPT // CHUNK):
                pltpu.sync_copy(vbuf,
                                a.at[pl.ds(sid * RPT + b * CHUNK, CHUNK)])
        for j in range(4):
            pltpu.sync_copy(idx_hbm.at[j, pl.ds(wid * nch, nch)], idxs[j])
        plsc.subcore_barrier()
        _fill_vec(vbuf, CHUNK, 1.0)

        def body(t, c):
            for j in range(4):
                pltpu.async_copy(vbuf, accs[j].at[idxs[j].at[t]],
                                 sems[j], add=True)
            for j in range(4):
                pltpu.make_async_copy(vbuf, accs[j].at[idxs[j].at[t]],
                                      sems[j]).wait()
            return c

        lax.fori_loop(0, nch, body, None)
        plsc.subcore_barrier()
        for j, a in enumerate(accs):
            for b in range(RPT // CHUNK):
                sl = pl.ds(sid * RPT + b * CHUNK, CHUNK)
                pltpu.sync_copy(a.at[sl], vbuf)
                pltpu.sync_copy(vbuf, out_hbm.at[cid, j, sl])

    return k(idx4)


def _segsum_job(tbl, src, dst, out, jslot, ctx):
    cid, sid, sidx, didx, rows, acc, semg, semi = ctx
    nch_all = src.shape[0] // NSUB
    n0 = max(2, 2 * int(round(nch_all * C0_SHARE / 2)))
    n1 = nch_all - n0
    npair = jnp.where(cid == 0, n0 // 2, n1 // 2)
    base = jnp.where(cid == 0, sid * n0, NSUB * n0 + sid * n1)
    r0 = sid * RPT

    def fire_idx(t, b):
        pltpu.async_copy(src.at[base + t], sidx.at[b], semi[b])
        pltpu.async_copy(dst.at[base + t], didx.at[b], semi[b])

    def wait_idx(t, b):
        pltpu.make_async_copy(src.at[base + t], sidx.at[b],
                              semi[b]).wait()
        pltpu.make_async_copy(dst.at[base + t], didx.at[b],
                              semi[b]).wait()

    def fire_g(b):
        pltpu.async_copy(tbl.at[sidx.at[b]], rows[b], semg[b])

    def wait_g(b):
        pltpu.make_async_copy(tbl.at[sidx.at[b]], rows[b], semg[b]).wait()

    _zero_rows(rows[0])
    for b in range(RPT // CHUNK):
        pltpu.sync_copy(rows[0], acc.at[pl.ds(r0 + b * CHUNK, CHUNK)])
    fire_idx(0, 0)
    fire_idx(1, 1)
    plsc.subcore_barrier()
    wait_idx(0, 0)
    fire_g(0)

    def body(p, c):
        k1 = 2 * p + 1
        wait_idx(k1, 1)
        fire_g(1)
        wait_g(0)
        pltpu.sync_copy(rows[0], acc.at[didx.at[0]], add=True)

        @pl.when(p < npair - 1)
        def _a():
            fire_idx(k1 + 1, 0)
            wait_idx(k1 + 1, 0)
            fire_g(0)

        wait_g(1)
        pltpu.sync_copy(rows[1], acc.at[didx.at[1]], add=True)

        @pl.when(p < npair - 1)
        def _b():
            fire_idx(k1 + 2, 1)

        return c

    lax.fori_loop(0, npair, body, None)
    plsc.subcore_barrier()
    for b in range(RPT // CHUNK):
        sl = pl.ds(r0 + b * CHUNK, CHUNK)
        pltpu.sync_copy(acc.at[sl], rows[0])
        pltpu.sync_copy(rows[0], out.at[jslot, cid, sl])
    plsc.subcore_barrier()


def _sc_segsum_multi(tables, edge_pairs, job_edges):
    njobs = len(tables)

    @functools.partial(
        pl.kernel, mesh=_mesh(),
        out_type=jax.ShapeDtypeStruct((njobs, NCORE, N_PAD, D), jnp.float32),
        scratch_types=[
            pltpu.VMEM((2, CHUNK), jnp.int32),
            pltpu.VMEM((2, CHUNK), jnp.int32),
            [pltpu.VMEM((CHUNK, D), jnp.float32) for _ in range(2)],
            pltpu.VMEM_SHARED((N_PAD, D), jnp.float32),
            [pltpu.SemaphoreType.DMA for _ in range(2)],
            [pltpu.SemaphoreType.DMA for _ in range(2)],
        ],
    )
    def k(*refs):
        tbls = refs[:njobs]
        epairs = refs[njobs:njobs + 2 * len(edge_pairs)]
        out = refs[njobs + 2 * len(edge_pairs)]
        sidx, didx, rows, acc, semg, semi = refs[njobs + 2 * len(edge_pairs) + 1:]
        cid = lax.axis_index("c")
        sid = lax.axis_index("s")
        ctx = (cid, sid, sidx, didx, rows, acc, semg, semi)
        for j in range(njobs):
            e = job_edges[j]
            _segsum_job(tbls[j], epairs[2 * e], epairs[2 * e + 1], out, j, ctx)

    flat_edges = []
    for s, d in edge_pairs:
        flat_edges += [s, d]
    return k(*tables, *flat_edges)


def _sc_gat(feat, el, er, src2d, dst2d):
    nch = src2d.shape[0] // NW
    npair = nch // 2

    @functools.partial(
        pl.kernel, mesh=_mesh(),
        out_type=(jax.ShapeDtypeStruct((NCORE, N_PAD, D), jnp.float32),
                  jax.ShapeDtypeStruct((NCORE, N_PAD), jnp.float32)),
        scratch_types=[
            pltpu.VMEM((2, CHUNK), jnp.int32),
            pltpu.VMEM((2, CHUNK), jnp.int32),
            [pltpu.VMEM((CHUNK,), jnp.float32) for _ in range(2)],
            [pltpu.VMEM((CHUNK,), jnp.float32) for _ in range(2)],
            pltpu.VMEM((CHUNK,), jnp.float32),
            [pltpu.VMEM((CHUNK, D), jnp.float32) for _ in range(2)],
            pltpu.VMEM_SHARED((N_PAD, D), jnp.float32),
            pltpu.VMEM_SHARED((N_PAD,), jnp.float32),
            [pltpu.SemaphoreType.DMA for _ in range(2)],
            [pltpu.SemaphoreType.DMA for _ in range(2)],
            [pltpu.SemaphoreType.DMA for _ in range(2)],
            [pltpu.SemaphoreType.DMA for _ in range(2)],
        ],
    )
    def k(feat_h, el_h, er_h, src, dst, out_rows, out_s,
          sidx, didx, elv, erv, wv, rows, acc, sacc, seml, semr, semf, semi):
        cid = lax.axis_index("c")
        sid = lax.axis_index("s")
        wid = cid * NSUB + sid
        r0 = sid * RPT

        def fire_idx(t, b):
            pltpu.async_copy(src.at[wid * nch + t], sidx.at[b], semi[b])
            pltpu.async_copy(dst.at[wid * nch + t], didx.at[b], semi[b])

        def wait_idx(t, b):
            pltpu.make_async_copy(src.at[wid * nch + t], sidx.at[b],
                                  semi[b]).wait()
            pltpu.make_async_copy(dst.at[wid * nch + t], didx.at[b],
                                  semi[b]).wait()

        def fire_g(b):
            pltpu.async_copy(el_h.at[sidx.at[b]], elv[b], seml[b])
            pltpu.async_copy(er_h.at[didx.at[b]], erv[b], semr[b])
            pltpu.async_copy(feat_h.at[sidx.at[b]], rows[b], semf[b])

        def work(b):
            pltpu.make_async_copy(el_h.at[sidx.at[b]], elv[b], seml[b]).wait()
            pltpu.make_async_copy(er_h.at[didx.at[b]], erv[b], semr[b]).wait()
            for c in range(CHUNK // 16):
                sl = pl.ds(c * 16, 16)
                e = elv[b][sl] + erv[b][sl]
                wv[sl] = jnp.exp(jnp.maximum(e, 0.2 * e))
            pltpu.sync_copy(wv, sacc.at[didx.at[b]], add=True)
            pltpu.make_async_copy(feat_h.at[sidx.at[b]], rows[b],
                                  semf[b]).wait()

            def wgrp(g, c2):
                wg = wv[pl.ds(g * 16, 16)]
                for j in range(16):
                    r = g * 16 + j
                    spl = _splat_lane(wg, j)
                    for col in range(D // 16):
                        sl = pl.ds(col * 16, 16)
                        rows[b][r, sl] = rows[b][r, sl] * spl
                return c2

            lax.fori_loop(0, CHUNK // 16, wgrp, None)
            pltpu.sync_copy(rows[b], acc.at[didx.at[b]], add=True)

        _zero_rows(rows[0])
        _fill_vec(wv, CHUNK, 0.0)
        for b in range(RPT // CHUNK):
            pltpu.sync_copy(rows[0], acc.at[pl.ds(r0 + b * CHUNK, CHUNK)])
            pltpu.sync_copy(wv, sacc.at[pl.ds(r0 + b * CHUNK, CHUNK)])
        fire_idx(0, 0)
        fire_idx(1, 1)
        plsc.subcore_barrier()
        wait_idx(0, 0)
        fire_g(0)

        def body(p, c):
            k1 = 2 * p + 1
            wait_idx(k1, 1)
            fire_g(1)
            work(0)

            @pl.when(p < npair - 1)
            def _a():
                fire_idx(k1 + 1, 0)
                wait_idx(k1 + 1, 0)
                fire_g(0)

            work(1)

            @pl.when(p < npair - 1)
            def _b():
                fire_idx(k1 + 2, 1)

            return c

        lax.fori_loop(0, npair, body, None)
        plsc.subcore_barrier()
        for b in range(RPT // CHUNK):
            sl = pl.ds(r0 + b * CHUNK, CHUNK)
            pltpu.sync_copy(acc.at[sl], rows[0])
            pltpu.sync_copy(rows[0], out_rows.at[cid, sl])
            pltpu.sync_copy(sacc.at[sl], wv)
            pltpu.sync_copy(wv, out_s.at[cid, sl])

    return k(feat, el, er, src2d, dst2d)


def _dot_rows(ra, rb, obuf):
    lanes = jnp.arange(16, dtype=jnp.int32)

    def g_body(g, c):
        res = jnp.zeros((16,), jnp.float32)
        for j in range(16):
            r = g * 16 + j
            accv = jnp.zeros((16,), jnp.float32)
            for col in range(D // 16):
                sl = pl.ds(col * 16, 16)
                accv = accv + ra[r, sl] * rb[r, sl]
            res = jnp.where(lanes == j, _hsum16(accv), res)
        obuf[pl.ds(g * 16, 16)] = res
        return c

    lax.fori_loop(0, CHUNK // 16, g_body, None)


def _sc_dots2(xu, xi, au, ai, src2d, dst2d):
    nch = src2d.shape[0] // NW
    npair = nch // 2
    e_pad = NW * nch * CHUNK

    @functools.partial(
        pl.kernel, mesh=_mesh(),
        out_type=(jax.ShapeDtypeStruct((e_pad,), jnp.float32),
                  jax.ShapeDtypeStruct((e_pad,), jnp.float32)),
        scratch_types=[
            pltpu.VMEM((2, CHUNK), jnp.int32),
            pltpu.VMEM((2, CHUNK), jnp.int32),
            [pltpu.VMEM((CHUNK, D), jnp.float32) for _ in range(2)],
            [pltpu.VMEM((CHUNK, D), jnp.float32) for _ in range(2)],
            pltpu.VMEM((CHUNK, D), jnp.float32),
            pltpu.VMEM((CHUNK, D), jnp.float32),
            pltpu.VMEM((CHUNK,), jnp.float32),
            [pltpu.SemaphoreType.DMA for _ in range(2)],
            pltpu.SemaphoreType.DMA,
            [pltpu.SemaphoreType.DMA for _ in range(2)],
        ],
    )
    def k(xu_h, xi_h, au_h, ai_h, src, dst, out_p, out_a,
          sidx, didx, pa, pb, aa, ab, obuf, semp, sema, semi):
        cid = lax.axis_index("c")
        sid = lax.axis_index("s")
        wid = cid * NSUB + sid

        def fire_idx(t, b):
            pltpu.async_copy(src.at[wid * nch + t], sidx.at[b], semi[b])
            pltpu.async_copy(dst.at[wid * nch + t], didx.at[b], semi[b])

        def wait_idx(t, b):
            pltpu.make_async_copy(src.at[wid * nch + t], sidx.at[b],
                                  semi[b]).wait()
            pltpu.make_async_copy(dst.at[wid * nch + t], didx.at[b],
                                  semi[b]).wait()

        def fire_pos(b):
            pltpu.async_copy(xu_h.at[sidx.at[b]], pa[b], semp[b])
            pltpu.async_copy(xi_h.at[didx.at[b]], pb[b], semp[b])

        def work_pos(t, b):
            pltpu.make_async_copy(xu_h.at[sidx.at[b]], pa[b], semp[b]).wait()
            pltpu.make_async_copy(xi_h.at[didx.at[b]], pb[b], semp[b]).wait()
            _dot_rows(pa[b], pb[b], obuf)
            pltpu.sync_copy(obuf, out_p.at[pl.ds(wid * nch * CHUNK + t * CHUNK,
                                                 CHUNK)])

        def fire_att(b):
            pltpu.async_copy(au_h.at[sidx.at[b]], aa, sema)
            pltpu.async_copy(ai_h.at[didx.at[b]], ab, sema)

        def work_att(t, b):
            pltpu.make_async_copy(au_h.at[sidx.at[b]], aa, sema).wait()
            pltpu.make_async_copy(ai_h.at[didx.at[b]], ab, sema).wait()
            _dot_rows(aa, ab, obuf)
            pltpu.sync_copy(obuf, out_a.at[pl.ds(wid * nch * CHUNK + t * CHUNK,
                                                 CHUNK)])

        fire_idx(0, 0)
        fire_idx(1, 1)
        wait_idx(0, 0)
        fire_pos(0)

        def body(p, c):
            k0 = 2 * p
            k1 = k0 + 1
            wait_idx(k1, 1)
            fire_pos(1)
            fire_att(0)
            work_pos(k0, 0)
            work_att(k0, 0)

            @pl.when(p < npair - 1)
            def _a():
                fire_idx(k1 + 1, 0)
                wait_idx(k1 + 1, 0)
                fire_pos(0)

            fire_att(1)
            work_pos(k1, 1)
            work_att(k1, 1)

            @pl.when(p < npair - 1)
            def _b():
                fire_idx(k1 + 2, 1)

            return c

        lax.fori_loop(0, npair, body, None)

    return k(xu, xi, au, ai, src2d, dst2d)


def _sc_dots1(ta, src2d, dst2d):
    nch = src2d.shape[0] // NW
    npair = nch // 2
    e_pad = NW * nch * CHUNK

    @functools.partial(
        pl.kernel, mesh=_mesh(),
        out_type=jax.ShapeDtypeStruct((e_pad,), jnp.float32),
        scratch_types=[
            pltpu.VMEM((2, CHUNK), jnp.int32),
            pltpu.VMEM((2, CHUNK), jnp.int32),
            [pltpu.VMEM((CHUNK, D), jnp.float32) for _ in range(2)],
            [pltpu.VMEM((CHUNK, D), jnp.float32) for _ in range(2)],
            pltpu.VMEM((CHUNK,), jnp.float32),
            [pltpu.SemaphoreType.DMA for _ in range(2)],
            [pltpu.SemaphoreType.DMA for _ in range(2)],
        ],
    )
    def k(t_h, src, dst, out_t, sidx, didx, ra, rb, obuf, semg, semi):
        cid = lax.axis_index("c")
        sid = lax.axis_index("s")
        wid = cid * NSUB + sid

        def fire_idx(t, b):
            pltpu.async_copy(src.at[wid * nch + t], sidx.at[b], semi[b])
            pltpu.async_copy(dst.at[wid * nch + t], didx.at[b], semi[b])

        def wait_idx(t, b):
            pltpu.make_async_copy(src.at[wid * nch + t], sidx.at[b],
                                  semi[b]).wait()
            pltpu.make_async_copy(dst.at[wid * nch + t], didx.at[b],
                                  semi[b]).wait()

        def fire_g(b):
            pltpu.async_copy(t_h.at[sidx.at[b]], ra[b], semg[b])
            pltpu.async_copy(t_h.at[didx.at[b]], rb[b], semg[b])

        def work(t, b):
            pltpu.make_async_copy(t_h.at[sidx.at[b]], ra[b], semg[b]).wait()
            pltpu.make_async_copy(t_h.at[didx.at[b]], rb[b], semg[b]).wait()
            _dot_rows(ra[b], rb[b], obuf)
            pltpu.sync_copy(obuf, out_t.at[pl.ds(wid * nch * CHUNK + t * CHUNK,
                                                 CHUNK)])

        fire_idx(0, 0)
        fire_idx(1, 1)
        wait_idx(0, 0)
        fire_g(0)

        def body(p, c):
            k1 = 2 * p + 1
            wait_idx(k1, 1)
            fire_g(1)
            work(k1 - 1, 0)

            @pl.when(p < npair - 1)
            def _a():
                fire_idx(k1 + 1, 0)
                wait_idx(k1 + 1, 0)
                fire_g(0)

            work(k1, 1)

            @pl.when(p < npair - 1)
            def _b():
                fire_idx(k1 + 2, 1)

            return c

        lax.fori_loop(0, npair, body, None)

    return k(ta, src2d, dst2d)


def _rb_spec():
    return pl.BlockSpec((ROW_BLK, D), lambda i: (i, 0))


def _col_spec():
    return pl.BlockSpec((ROW_BLK, 1), lambda i: (i, 0))


def _w_spec():
    return pl.BlockSpec((D, D), lambda i: (0, 0))


def _p_spec():
    return pl.BlockSpec((NCORE, ROW_BLK, D), lambda i: (0, i, 0))


def _tc_stage1(ue, ie, te, au, ai, degp, W_r1, W_rb1, W_t1, W_a1r, W_a1rb,
               W_gat, al, ar):
    def body(ue_r, ie_r, te_r, au_r, ai_r, dp_r, wr1, wrb1, wt1, wa1r, wa1rb,
             wg, al_r, ar_r,
             t1_o, t2_o, t3_o, t4_o, t5_o, feat_o, el_o, er_o,
             frs_o, frd_o, fts_o, ftd_o, fss_o, fsd_o):
        dp = dp_r[...]
        dru = dp[0, 0] + dp[1, 0]
        dri = dp[0, 1] + dp[1, 1]
        dts = dp[0, 2] + dp[1, 2]
        dtd = dp[0, 3] + dp[1, 3]
        frs = lax.rsqrt(jnp.maximum(dru, 1.0))
        frd = lax.rsqrt(jnp.maximum(dri, 1.0))
        fts = lax.rsqrt(jnp.maximum(dts, 1.0))
        ftd = lax.rsqrt(jnp.maximum(dtd, 1.0))
        fss = lax.rsqrt(dts + 1.0)
        fsd = lax.rsqrt(dtd + 1.0)
        frs_o[...] = frs[:, None]
        frd_o[...] = frd[:, None]
        fts_o[...] = fts[:, None]
        ftd_o[...] = ftd[:, None]
        fss_o[...] = fss[:, None]
        fsd_o[...] = fsd[:, None]
        ue_b = ue_r[...]
        ie_b = ie_r[...]
        te_b = te_r[...]
        t1_o[...] = jnp.dot(ue_b, wr1[...], preferred_element_type=jnp.float32) * frs[:, None]
        t2_o[...] = jnp.dot(ie_b, wrb1[...], preferred_element_type=jnp.float32) * frd[:, None]
        t3_o[...] = jnp.dot(ue_b, wt1[...], preferred_element_type=jnp.float32) * fts[:, None]
        t4_o[...] = jnp.dot(au_r[...], wa1r[...], preferred_element_type=jnp.float32) * frs[:, None]
        t5_o[...] = jnp.dot(ai_r[...], wa1rb[...], preferred_element_type=jnp.float32) * frd[:, None]
        feat = jnp.dot(te_b, wg[...], preferred_element_type=jnp.float32)
        feat_o[...] = feat
        el_o[...] = jnp.dot(feat, al_r[...], preferred_element_type=jnp.float32)
        er_o[...] = jnp.dot(feat, ar_r[...], preferred_element_type=jnp.float32)

    rb = jax.ShapeDtypeStruct((N_PAD, D), jnp.float32)
    col = jax.ShapeDtypeStruct((N_PAD, 1), jnp.float32)
    return pl.pallas_call(
        body,
        grid=(GRID,),
        in_specs=[_rb_spec()] * 5
        + [pl.BlockSpec((NCORE, 4, ROW_BLK), lambda i: (0, 0, i))]
        + [_w_spec()] * 6
        + [pl.BlockSpec((D, 1), lambda i: (0, 0))] * 2,
        out_specs=[_rb_spec()] * 6 + [_col_spec()] * 2 + [_col_spec()] * 6,
        out_shape=[rb] * 6 + [col] * 8,
    )(ue, ie, te, au, ai, degp, W_r1, W_rb1, W_t1, W_a1r, W_a1rb, W_gat, al, ar)


def _tc_stage2(P1, P2, P3, P4, P5, Pg, Ps, frs, frd, fts, ftd, fss, fsd,
               W_r2, W_rb2, W_t2, W_a2r, W_a2rb, W_tg2):
    def body(p1, p2, p3, p4, p5, pg, ps, frs_r, frd_r, fts_r, ftd_r, fss_r,
             fsd_r, wr2, wrb2, wt2, wa2r, wa2rb, wtg2,
             t7_o, t8_o, t9_o, t10_o, t11_o, t12_o, xiid1_o, aiid1_o):
        frs_b = frs_r[...]
        frd_b = frd_r[...]
        x_iid1 = _leaky((p1[0] + p1[1]) * frd_b)
        x_uid1 = 0.5 * (_leaky((p2[0] + p2[1]) * frs_b)
                        + _leaky((p3[0] + p3[1]) * ftd_r[...]))
        a_iid1 = _leaky((p4[0] + p4[1]) * frd_b)
        a_uid1 = _leaky((p5[0] + p5[1]) * frs_b)
        s = ps[0] + ps[1]
        gat1 = _leaky((pg[0] + pg[1]) / (s[:, None] + 1e-9))
        t7_o[...] = jnp.dot(x_uid1, wr2[...], preferred_element_type=jnp.float32) * frs_b
        t8_o[...] = jnp.dot(x_iid1, wrb2[...], preferred_element_type=jnp.float32) * frd_b
        t9_o[...] = jnp.dot(x_uid1, wt2[...], preferred_element_type=jnp.float32) * fts_r[...]
        t10_o[...] = jnp.dot(a_uid1, wa2r[...], preferred_element_type=jnp.float32) * frs_b
        t11_o[...] = jnp.dot(a_iid1, wa2rb[...], preferred_element_type=jnp.float32) * frd_b
        t12_o[...] = jnp.dot(gat1, wtg2[...], preferred_element_type=jnp.float32) * fss_r[...]
        xiid1_o[...] = x_iid1
        aiid1_o[...] = a_iid1

    rb = jax.ShapeDtypeStruct((N_PAD, D), jnp.float32)
    return pl.pallas_call(
        body,
        grid=(GRID,),
        in_specs=[_p_spec()] * 6
        + [pl.BlockSpec((NCORE, ROW_BLK), lambda i: (0, i))]
        + [_col_spec()] * 6 + [_w_spec()] * 6,
        out_specs=[_rb_spec()] * 8,
        out_shape=[rb] * 8,
    )(P1, P2, P3, P4, P5, Pg, Ps, frs, frd, fts, ftd, fss, fsd,
      W_r2, W_rb2, W_t2, W_a2r, W_a2rb, W_tg2)


def _tc_stage3(P7, P8, P9, P10, P11, P12, frs, frd, ftd, fsd):
    def body(p7, p8, p9, p10, p11, p12, frs_r, frd_r, ftd_r, fsd_r,
             xu_o, xi_o, au_o, ai_o, t_o):
        frs_b = frs_r[...]
        frd_b = frd_r[...]
        xi_o[...] = _leaky((p7[0] + p7[1]) * frd_b)
        xu_o[...] = jnp.maximum(_leaky((p8[0] + p8[1]) * frs_b),
                                _leaky((p9[0] + p9[1]) * ftd_r[...]))
        ai_o[...] = _leaky((p10[0] + p10[1]) * frd_b)
        au_o[...] = _leaky((p11[0] + p11[1]) * frs_b)
        t_o[...] = _leaky((p12[0] + p12[1]) * fsd_r[...])

    rb = jax.ShapeDtypeStruct((N_PAD, D), jnp.float32)
    return pl.pallas_call(
        body,
        grid=(GRID,),
        in_specs=[_p_spec()] * 6 + [_col_spec()] * 4,
        out_specs=[_rb_spec()] * 5,
        out_shape=[rb] * 5,
    )(P7, P8, P9, P10, P11, P12, frs, frd, ftd, fsd)


def _tc_edge_losses(ratings2d, pos2d, att2d, tr2d):
    rows = ratings2d.shape[0]
    blk = rows

    def body(rt_r, po_r, at_r, tr_r, pos_o, sums_o):
        rt = rt_r[...]
        po = po_r[...]
        err = rt - (po + MEAN_RATE)
        pos_o[...] = po + MEAN_RATE
        at = at_r[...]
        att_s = 1.0 / (1.0 + jnp.exp(-at))
        tgt = 1.0 / (1.0 + jnp.exp(-(rt - MEAN_RATE)))
        tr = tr_r[...]
        sp = jnp.maximum(-tr, 0.0) + jnp.log(1.0 + jnp.exp(-jnp.abs(tr)))
        sg = 1.0 / (1.0 + jnp.exp(-tr))
        upd = jnp.stack([
            jnp.sum(err * err, axis=0),
            jnp.sum(jnp.abs(err), axis=0),
            jnp.sum((att_s - tgt) ** 2, axis=0),
            jnp.sum(sp, axis=0),
            jnp.sum(sg, axis=0),
            jnp.zeros((D,), jnp.float32),
            jnp.zeros((D,), jnp.float32),
            jnp.zeros((D,), jnp.float32),
        ])
        sums_o[...] = upd

    return pl.pallas_call(
        body,
        grid=(1,),
        in_specs=[pl.BlockSpec((blk, D), lambda i: (i, 0))] * 4,
        out_specs=[pl.BlockSpec((blk, D), lambda i: (i, 0)),
                   pl.BlockSpec((8, D), lambda i: (0, 0))],
        out_shape=[jax.ShapeDtypeStruct((rows, D), jnp.float32),
                   jax.ShapeDtypeStruct((8, D), jnp.float32)],
    )(ratings2d, pos2d, att2d, tr2d)


def _tc_table_sums(xu, xi, t, au, ai):
    def body(xu_r, xi_r, t_r, au_r, ai_r, sums_o):
        i = pl.program_id(0)

        @pl.when(i == 0)
        def _():
            sums_o[...] = jnp.zeros((8, D), jnp.float32)

        xi_b = xi_r[...]
        ai_b = ai_r[...]
        reg = (jnp.sum(jnp.abs(xu_r[...]), axis=0) + jnp.sum(jnp.abs(xi_b), axis=0)
               + jnp.sum(jnp.abs(t_r[...]), axis=0) + jnp.sum(jnp.abs(au_r[...]), axis=0)
               + jnp.sum(jnp.abs(ai_b), axis=0))
        ax = jnp.sum(jnp.abs(xi_b - ai_b), axis=0)
        z = jnp.zeros((D,), jnp.float32)
        sums_o[...] = sums_o[...] + jnp.stack([reg, ax, z, z, z, z, z, z])

    return pl.pallas_call(
        body,
        grid=(GRID,),
        in_specs=[_rb_spec()] * 5,
        out_specs=pl.BlockSpec((8, D), lambda i: (0, 0)),
        out_shape=jax.ShapeDtypeStruct((8, D), jnp.float32),
    )(xu, xi, t, au, ai)


def kernel(user_emb, item_emb, trust_emb, a_emb_uid, a_emb_iid, ratings,
           rated_edge_index, trust_edge_index,
           W_r1, W_rb1, W_t1, W_r2, W_rb2, W_t2,
           W_a1r, W_a1rb, W_a2r, W_a2rb, W_gat, W_tg2, attn_l, attn_r):
    f32 = jnp.float32

    def pad_rows(x):
        return jnp.concatenate([x, jnp.zeros((N_PAD - x.shape[0], D), f32)])

    ue = pad_rows(user_emb)
    ie = pad_rows(item_emb)
    te = pad_rows(trust_emb)
    au = pad_rows(a_emb_uid)
    ai = pad_rows(a_emb_iid)

    rs = rated_edge_index[0].astype(jnp.int32)
    rd = rated_edge_index[1].astype(jnp.int32)
    ts = trust_edge_index[0].astype(jnp.int32)
    td = trust_edge_index[1].astype(jnp.int32)

    grain = NW * CHUNK * 4
    e_pad_r = ((E_R + grain - 1) // grain) * grain
    rs_p = _pad_edges(rs, e_pad_r, PAD_SRC).reshape(-1, CHUNK)
    rd_p = _pad_edges(rd, e_pad_r, PAD_DST).reshape(-1, CHUNK)
    ts_p = _pad_edges(ts, e_pad_r, PAD_SRC).reshape(-1, CHUNK)
    td_p = _pad_edges(td, e_pad_r, PAD_DST).reshape(-1, CHUNK)

    sl = jnp.arange(N_U, dtype=jnp.int32)
    tsl_s = jnp.concatenate([ts, sl])
    tsl_d = jnp.concatenate([td, sl])
    e_pad_t = ((tsl_s.shape[0] + grain - 1) // grain) * grain
    tsl_s_p = _pad_edges(tsl_s, e_pad_t, PAD_SRC).reshape(-1, CHUNK)
    tsl_d_p = _pad_edges(tsl_d, e_pad_t, PAD_DST).reshape(-1, CHUNK)

    idx4 = jnp.stack([rs_p, rd_p, ts_p, td_p])
    degp = _sc_degrees(idx4)


    (tb_r1, tb_rb1, tb_t1, tb_a1r, tb_a1rb, feat,
     el2, er2, frs, frd, fts, ftd, fss, fsd) = _tc_stage1(
        ue, ie, te, au, ai, degp, W_r1, W_rb1, W_t1, W_a1r, W_a1rb, W_gat,
        attn_l.reshape(D, 1), attn_r.reshape(D, 1))

    PA = _sc_segsum_multi(
        [tb_r1, tb_rb1, tb_t1, tb_a1r, tb_a1rb],
        [(rs_p, rd_p), (rd_p, rs_p), (ts_p, td_p)],
        [0, 1, 2, 0, 1])
    P1, P2, P3, P4, P5 = (PA[j] for j in range(5))
    Pg, Ps = _sc_gat(feat, el2.reshape(N_PAD), er2.reshape(N_PAD),
                     tsl_s_p, tsl_d_p)

    (tb_r2, tb_rb2, tb_t2, tb_a2r, tb_a2rb, tb_tg2,
     x_iid1, a_iid1) = _tc_stage2(
        P1, P2, P3, P4, P5, Pg, Ps, frs, frd, fts, ftd, fss, fsd,
        W_r2, W_rb2, W_t2, W_a2r, W_a2rb, W_tg2)
    del x_iid1, a_iid1

    PB = _sc_segsum_multi(
        [tb_r2, tb_rb2, tb_t2, tb_a2r, tb_a2rb, tb_tg2],
        [(rs_p, rd_p), (rd_p, rs_p), (ts_p, td_p), (tsl_s_p, tsl_d_p)],
        [0, 1, 2, 0, 1, 3])
    P7, P8, P9, P10, P11, P12 = (PB[j] for j in range(6))

    x_uid, x_iid, a_uid, a_iid, t = _tc_stage3(
        P7, P8, P9, P10, P11, P12, frs, frd, ftd, fsd)

    pos_pre, att_pre = _sc_dots2(x_uid, x_iid, a_uid, a_iid, rs_p, rd_p)
    tr_pre = _sc_dots1(t, ts_p, td_p)

    ratings2d = ratings.reshape(E_R // D, D)
    pos2d = pos_pre[:E_R].reshape(E_R // D, D)
    att2d = att_pre[:E_R].reshape(E_R // D, D)
    tr2d = tr_pre[:E_T].reshape(E_T // D, D)
    pos_out2d, esums = _tc_edge_losses(ratings2d, pos2d, att2d, tr2d)
    tsums = _tc_table_sums(x_uid, x_iid, t, a_uid, a_iid)

    rating_loss = jnp.sum(esums[0]) / E_R
    mae = jnp.sum(esums[1]) / E_R
    l_att = jnp.sum(esums[2]) / E_R
    loss_trust = jnp.sum(esums[3]) / E_T
    trust_auc = jnp.sum(esums[4]) / E_T
    trust_ap = trust_auc
    loss_reg = jnp.sum(tsums[0])
    loss_a_x = jnp.sum(tsums[1])
    pos_score = pos_out2d.reshape(E_R)

    return (rating_loss, mae, loss_reg, pos_score, l_att, loss_a_x,
            trust_auc, trust_ap, loss_trust)


# speedup vs baseline: 1.0929x; 1.0929x over previous
import functools

import jax
import jax.numpy as jnp
from jax import lax
from jax.experimental import pallas as pl
from jax.experimental.pallas import tpu as pltpu
from jax.experimental.pallas import tpu_sc as plsc

N_U = 10000
N_I = 10000
D = 128
E_R = 320000
E_T = 320000
MEAN_RATE = 3.5

N_PAD = 10240
NSUB = 16
NCORE = 2
NW = NCORE * NSUB
RPT = N_PAD // NSUB
CHUNK = 128
PAD_SRC = N_U
PAD_DST = 10200
ROW_BLK = 512
GRID = N_PAD // ROW_BLK
C0_SHARE = 0.65


def _leaky(x):
    return jnp.maximum(x, 0.01 * x)


def _mesh():
    return plsc.VectorSubcoreMesh(core_axis_name="c", subcore_axis_name="s")


_GDN = lax.GatherDimensionNumbers(
    offset_dims=(), collapsed_slice_dims=(0,), start_index_map=(0,))


def _splat_lane(vec16, j):
    idx = jnp.full((16, 1), j, jnp.int32)
    return lax.gather(vec16, idx, _GDN, slice_sizes=(1,),
                      mode=lax.GatherScatterMode.PROMISE_IN_BOUNDS)


def _shuffle(vec16, idx):
    return lax.gather(vec16, idx[:, None], _GDN, slice_sizes=(1,),
                      mode=lax.GatherScatterMode.PROMISE_IN_BOUNDS)


def _hsum16(x):
    lanes = jnp.arange(16, dtype=jnp.int32)
    for off in (8, 4, 2, 1):
        x = x + _shuffle(x, lanes ^ off)
    return x


def _fill_vec(ref, n, val):
    v = jnp.full((16,), val, jnp.float32)

    def zb(i, c):
        ref[pl.ds(i * 16, 16)] = v
        return c

    lax.fori_loop(0, n // 16, zb, None)


def _zero_rows(ref):
    z = jnp.zeros((16,), jnp.float32)

    def zb(r, c):
        for col in range(D // 16):
            ref[r, pl.ds(col * 16, 16)] = z
        return c

    lax.fori_loop(0, CHUNK, zb, None)


def _pad_edges(idx, e_pad, fill):
    return jnp.concatenate(
        [idx.astype(jnp.int32), jnp.full((e_pad - idx.shape[0],), fill, jnp.int32)])


def _sc_degrees(idx4):
    nch = idx4.shape[1] // NW

    @functools.partial(
        pl.kernel, mesh=_mesh(),
        out_type=jax.ShapeDtypeStruct((NCORE, 4, N_PAD), jnp.float32),
        scratch_types=[
            [pltpu.VMEM((nch, CHUNK), jnp.int32) for _ in range(4)],
            pltpu.VMEM((CHUNK,), jnp.float32),
            [pltpu.VMEM_SHARED((N_PAD,), jnp.float32) for _ in range(4)],
            [pltpu.SemaphoreType.DMA for _ in range(4)],
        ],
    )
    def k(idx_hbm, out_hbm, idxs, vbuf, accs, sems):
        cid = lax.axis_index("c")
        sid = lax.axis_index("s")
        wid = cid * NSUB + sid
        _fill_vec(vbuf, CHUNK, 0.0)
        for a in accs:
            for b in range(RPT // CHUNK):
                pltpu.sync_copy(vbuf,
                                a.at[pl.ds(sid * RPT + b * CHUNK, CHUNK)])
        for j in range(4):
            pltpu.sync_copy(idx_hbm.at[j, pl.ds(wid * nch, nch)], idxs[j])
        plsc.subcore_barrier()
        _fill_vec(vbuf, CHUNK, 1.0)

        def body(t, c):
            for j in range(4):
                pltpu.async_copy(vbuf, accs[j].at[idxs[j].at[t]],
                                 sems[j], add=True)
            for j in range(4):
                pltpu.make_async_copy(vbuf, accs[j].at[idxs[j].at[t]],
                                      sems[j]).wait()
            return c

        lax.fori_loop(0, nch, body, None)
        plsc.subcore_barrier()
        for j, a in enumerate(accs):
            for b in range(RPT // CHUNK):
                sl = pl.ds(sid * RPT + b * CHUNK, CHUNK)
                pltpu.sync_copy(a.at[sl], vbuf)
                pltpu.sync_copy(vbuf, out_hbm.at[cid, j, sl])

    return k(idx4)


def _segsum_job(tbl, src, dst, out, jslot, ctx):
    cid, sid, sidx, didx, rows, acc, semg, semi = ctx
    nch_all = src.shape[0] // NSUB
    n0 = max(2, 2 * int(round(nch_all * C0_SHARE / 2)))
    n1 = nch_all - n0
    npair = jnp.where(cid == 0, n0 // 2, n1 // 2)
    base = jnp.where(cid == 0, sid * n0, NSUB * n0 + sid * n1)
    r0 = sid * RPT

    def fire_idx(t, b):
        pltpu.async_copy(src.at[base + t], sidx.at[b], semi[b])
        pltpu.async_copy(dst.at[base + t], didx.at[b], semi[b])

    def wait_idx(t, b):
        pltpu.make_async_copy(src.at[base + t], sidx.at[b],
                              semi[b]).wait()
        pltpu.make_async_copy(dst.at[base + t], didx.at[b],
                              semi[b]).wait()

    def fire_g(b):
        pltpu.async_copy(tbl.at[sidx.at[b]], rows[b], semg[b])

    def wait_g(b):
        pltpu.make_async_copy(tbl.at[sidx.at[b]], rows[b], semg[b]).wait()

    _zero_rows(rows[0])
    for b in range(RPT // CHUNK):
        pltpu.sync_copy(rows[0], acc.at[pl.ds(r0 + b * CHUNK, CHUNK)])
    fire_idx(0, 0)
    fire_idx(1, 1)
    plsc.subcore_barrier()
    wait_idx(0, 0)
    fire_g(0)

    def body(p, c):
        k1 = 2 * p + 1
        wait_idx(k1, 1)
        fire_g(1)
        wait_g(0)
        pltpu.sync_copy(rows[0], acc.at[didx.at[0]], add=True)

        @pl.when(p < npair - 1)
        def _a():
            fire_idx(k1 + 1, 0)
            wait_idx(k1 + 1, 0)
            fire_g(0)

        wait_g(1)
        pltpu.sync_copy(rows[1], acc.at[didx.at[1]], add=True)

        @pl.when(p < npair - 1)
        def _b():
            fire_idx(k1 + 2, 1)

        return c

    lax.fori_loop(0, npair, body, None)
    plsc.subcore_barrier()
    for b in range(RPT // CHUNK):
        sl = pl.ds(r0 + b * CHUNK, CHUNK)
        pltpu.sync_copy(acc.at[sl], rows[0])
        pltpu.sync_copy(rows[0], out.at[jslot, cid, sl])
    plsc.subcore_barrier()


def _sc_segsum_multi(tables, edge_pairs, job_edges):
    njobs = len(tables)

    @functools.partial(
        pl.kernel, mesh=_mesh(),
        out_type=jax.ShapeDtypeStruct((njobs, NCORE, N_PAD, D), jnp.float32),
        scratch_types=[
            pltpu.VMEM((2, CHUNK), jnp.int32),
            pltpu.VMEM((2, CHUNK), jnp.int32),
            [pltpu.VMEM((CHUNK, D), jnp.float32) for _ in range(2)],
            pltpu.VMEM_SHARED((N_PAD, D), jnp.float32),
            [pltpu.SemaphoreType.DMA for _ in range(2)],
            [pltpu.SemaphoreType.DMA for _ in range(2)],
        ],
    )
    def k(*refs):
        tbls = refs[:njobs]
        epairs = refs[njobs:njobs + 2 * len(edge_pairs)]
        out = refs[njobs + 2 * len(edge_pairs)]
        sidx, didx, rows, acc, semg, semi = refs[njobs + 2 * len(edge_pairs) + 1:]
        cid = lax.axis_index("c")
        sid = lax.axis_index("s")
        ctx = (cid, sid, sidx, didx, rows, acc, semg, semi)
        for j in range(njobs):
            e = job_edges[j]
            _segsum_job(tbls[j], epairs[2 * e], epairs[2 * e + 1], out, j, ctx)

    flat_edges = []
    for s, d in edge_pairs:
        flat_edges += [s, d]
    return k(*tables, *flat_edges)


def _sc_gat(feat, el, er, src2d, dst2d):
    nch = src2d.shape[0] // NW
    npair = nch // 2

    @functools.partial(
        pl.kernel, mesh=_mesh(),
        out_type=(jax.ShapeDtypeStruct((NCORE, N_PAD, D), jnp.float32),
                  jax.ShapeDtypeStruct((NCORE, N_PAD), jnp.float32)),
        scratch_types=[
            pltpu.VMEM((2, CHUNK), jnp.int32),
            pltpu.VMEM((2, CHUNK), jnp.int32),
            [pltpu.VMEM((CHUNK,), jnp.float32) for _ in range(2)],
            [pltpu.VMEM((CHUNK,), jnp.float32) for _ in range(2)],
            pltpu.VMEM((CHUNK,), jnp.float32),
            [pltpu.VMEM((CHUNK, D), jnp.float32) for _ in range(2)],
            pltpu.VMEM_SHARED((N_PAD, D), jnp.float32),
            pltpu.VMEM_SHARED((N_PAD,), jnp.float32),
            [pltpu.SemaphoreType.DMA for _ in range(2)],
            [pltpu.SemaphoreType.DMA for _ in range(2)],
            [pltpu.SemaphoreType.DMA for _ in range(2)],
            [pltpu.SemaphoreType.DMA for _ in range(2)],
        ],
    )
    def k(feat_h, el_h, er_h, src, dst, out_rows, out_s,
          sidx, didx, elv, erv, wv, rows, acc, sacc, seml, semr, semf, semi):
        cid = lax.axis_index("c")
        sid = lax.axis_index("s")
        wid = cid * NSUB + sid
        r0 = sid * RPT

        def fire_idx(t, b):
            pltpu.async_copy(src.at[wid * nch + t], sidx.at[b], semi[b])
            pltpu.async_copy(dst.at[wid * nch + t], didx.at[b], semi[b])

        def wait_idx(t, b):
            pltpu.make_async_copy(src.at[wid * nch + t], sidx.at[b],
                                  semi[b]).wait()
            pltpu.make_async_copy(dst.at[wid * nch + t], didx.at[b],
                                  semi[b]).wait()

        def fire_g(b):
            pltpu.async_copy(el_h.at[sidx.at[b]], elv[b], seml[b])
            pltpu.async_copy(er_h.at[didx.at[b]], erv[b], semr[b])
            pltpu.async_copy(feat_h.at[sidx.at[b]], rows[b], semf[b])

        def work(b):
            pltpu.make_async_copy(el_h.at[sidx.at[b]], elv[b], seml[b]).wait()
            pltpu.make_async_copy(er_h.at[didx.at[b]], erv[b], semr[b]).wait()
            for c in range(CHUNK // 16):
                sl = pl.ds(c * 16, 16)
                e = elv[b][sl] + erv[b][sl]
                wv[sl] = jnp.exp(jnp.maximum(e, 0.2 * e))
            pltpu.sync_copy(wv, sacc.at[didx.at[b]], add=True)
            pltpu.make_async_copy(feat_h.at[sidx.at[b]], rows[b],
                                  semf[b]).wait()

            def wgrp(g, c2):
                wg = wv[pl.ds(g * 16, 16)]
                for j in range(16):
                    r = g * 16 + j
                    spl = _splat_lane(wg, j)
                    for col in range(D // 16):
                        sl = pl.ds(col * 16, 16)
                        rows[b][r, sl] = rows[b][r, sl] * spl
                return c2

            lax.fori_loop(0, CHUNK // 16, wgrp, None)
            pltpu.sync_copy(rows[b], acc.at[didx.at[b]], add=True)

        _zero_rows(rows[0])
        _fill_vec(wv, CHUNK, 0.0)
        for b in range(RPT // CHUNK):
            pltpu.sync_copy(rows[0], acc.at[pl.ds(r0 + b * CHUNK, CHUNK)])
            pltpu.sync_copy(wv, sacc.at[pl.ds(r0 + b * CHUNK, CHUNK)])
        fire_idx(0, 0)
        fire_idx(1, 1)
        plsc.subcore_barrier()
        wait_idx(0, 0)
        fire_g(0)

        def body(p, c):
            k1 = 2 * p + 1
            wait_idx(k1, 1)
            fire_g(1)
            work(0)

            @pl.when(p < npair - 1)
            def _a():
                fire_idx(k1 + 1, 0)
                wait_idx(k1 + 1, 0)
                fire_g(0)

            work(1)

            @pl.when(p < npair - 1)
            def _b():
                fire_idx(k1 + 2, 1)

            return c

        lax.fori_loop(0, npair, body, None)
        plsc.subcore_barrier()
        for b in range(RPT // CHUNK):
            sl = pl.ds(r0 + b * CHUNK, CHUNK)
            pltpu.sync_copy(acc.at[sl], rows[0])
            pltpu.sync_copy(rows[0], out_rows.at[cid, sl])
            pltpu.sync_copy(sacc.at[sl], wv)
            pltpu.sync_copy(wv, out_s.at[cid, sl])

    return k(feat, el, er, src2d, dst2d)


def _dot_rows(ra, rb, obuf):
    lanes = jnp.arange(16, dtype=jnp.int32)

    def g_body(g, c):
        res = jnp.zeros((16,), jnp.float32)
        for j in range(16):
            r = g * 16 + j
            accv = jnp.zeros((16,), jnp.float32)
            for col in range(D // 16):
                sl = pl.ds(col * 16, 16)
                accv = accv + ra[r, sl] * rb[r, sl]
            res = jnp.where(lanes == j, _hsum16(accv), res)
        obuf[pl.ds(g * 16, 16)] = res
        return c

    lax.fori_loop(0, CHUNK // 16, g_body, None)


def _sc_dots2(xu, xi, au, ai, src2d, dst2d):
    nch = src2d.shape[0] // NW
    npair = nch // 2
    e_pad = NW * nch * CHUNK

    @functools.partial(
        pl.kernel, mesh=_mesh(),
        out_type=(jax.ShapeDtypeStruct((e_pad,), jnp.float32),
                  jax.ShapeDtypeStruct((e_pad,), jnp.float32)),
        scratch_types=[
            pltpu.VMEM((2, CHUNK), jnp.int32),
            pltpu.VMEM((2, CHUNK), jnp.int32),
            [pltpu.VMEM((CHUNK, D), jnp.float32) for _ in range(2)],
            [pltpu.VMEM((CHUNK, D), jnp.float32) for _ in range(2)],
            pltpu.VMEM((CHUNK, D), jnp.float32),
            pltpu.VMEM((CHUNK, D), jnp.float32),
            pltpu.VMEM((CHUNK,), jnp.float32),
            [pltpu.SemaphoreType.DMA for _ in range(2)],
            pltpu.SemaphoreType.DMA,
            [pltpu.SemaphoreType.DMA for _ in range(2)],
        ],
    )
    def k(xu_h, xi_h, au_h, ai_h, src, dst, out_p, out_a,
          sidx, didx, pa, pb, aa, ab, obuf, semp, sema, semi):
        cid = lax.axis_index("c")
        sid = lax.axis_index("s")
        wid = cid * NSUB + sid

        def fire_idx(t, b):
            pltpu.async_copy(src.at[wid * nch + t], sidx.at[b], semi[b])
            pltpu.async_copy(dst.at[wid * nch + t], didx.at[b], semi[b])

        def wait_idx(t, b):
            pltpu.make_async_copy(src.at[wid * nch + t], sidx.at[b],
                                  semi[b]).wait()
            pltpu.make_async_copy(dst.at[wid * nch + t], didx.at[b],
                                  semi[b]).wait()

        def fire_pos(b):
            pltpu.async_copy(xu_h.at[sidx.at[b]], pa[b], semp[b])
            pltpu.async_copy(xi_h.at[didx.at[b]], pb[b], semp[b])

        def work_pos(t, b):
            pltpu.make_async_copy(xu_h.at[sidx.at[b]], pa[b], semp[b]).wait()
            pltpu.make_async_copy(xi_h.at[didx.at[b]], pb[b], semp[b]).wait()
            _dot_rows(pa[b], pb[b], obuf)
            pltpu.sync_copy(obuf, out_p.at[pl.ds(wid * nch * CHUNK + t * CHUNK,
                                                 CHUNK)])

        def fire_att(b):
            pltpu.async_copy(au_h.at[sidx.at[b]], aa, sema)
            pltpu.async_copy(ai_h.at[didx.at[b]], ab, sema)

        def work_att(t, b):
            pltpu.make_async_copy(au_h.at[sidx.at[b]], aa, sema).wait()
            pltpu.make_async_copy(ai_h.at[didx.at[b]], ab, sema).wait()
            _dot_rows(aa, ab, obuf)
            pltpu.sync_copy(obuf, out_a.at[pl.ds(wid * nch * CHUNK + t * CHUNK,
                                                 CHUNK)])

        fire_idx(0, 0)
        fire_idx(1, 1)
        wait_idx(0, 0)
        fire_pos(0)

        def body(p, c):
            k0 = 2 * p
            k1 = k0 + 1
            wait_idx(k1, 1)
            fire_pos(1)
            fire_att(0)
            work_pos(k0, 0)
            work_att(k0, 0)

            @pl.when(p < npair - 1)
            def _a():
                fire_idx(k1 + 1, 0)
                wait_idx(k1 + 1, 0)
                fire_pos(0)

            fire_att(1)
            work_pos(k1, 1)
            work_att(k1, 1)

            @pl.when(p < npair - 1)
            def _b():
                fire_idx(k1 + 2, 1)

            return c

        lax.fori_loop(0, npair, body, None)

    return k(xu, xi, au, ai, src2d, dst2d)


def _sc_dots1(ta, src2d, dst2d):
    nch = src2d.shape[0] // NW
    npair = nch // 2
    e_pad = NW * nch * CHUNK

    @functools.partial(
        pl.kernel, mesh=_mesh(),
        out_type=jax.ShapeDtypeStruct((e_pad,), jnp.float32),
        scratch_types=[
            pltpu.VMEM((2, CHUNK), jnp.int32),
            pltpu.VMEM((2, CHUNK), jnp.int32),
            [pltpu.VMEM((CHUNK, D), jnp.float32) for _ in range(2)],
            [pltpu.VMEM((CHUNK, D), jnp.float32) for _ in range(2)],
            pltpu.VMEM((CHUNK,), jnp.float32),
            [pltpu.SemaphoreType.DMA for _ in range(2)],
            [pltpu.SemaphoreType.DMA for _ in range(2)],
        ],
    )
    def k(t_h, src, dst, out_t, sidx, didx, ra, rb, obuf, semg, semi):
        cid = lax.axis_index("c")
        sid = lax.axis_index("s")
        wid = cid * NSUB + sid

        def fire_idx(t, b):
            pltpu.async_copy(src.at[wid * nch + t], sidx.at[b], semi[b])
            pltpu.async_copy(dst.at[wid * nch + t], didx.at[b], semi[b])

        def wait_idx(t, b):
            pltpu.make_async_copy(src.at[wid * nch + t], sidx.at[b],
                                  semi[b]).wait()
            pltpu.make_async_copy(dst.at[wid * nch + t], didx.at[b],
                                  semi[b]).wait()

        def fire_g(b):
            pltpu.async_copy(t_h.at[sidx.at[b]], ra[b], semg[b])
            pltpu.async_copy(t_h.at[didx.at[b]], rb[b], semg[b])

        def work(t, b):
            pltpu.make_async_copy(t_h.at[sidx.at[b]], ra[b], semg[b]).wait()
            pltpu.make_async_copy(t_h.at[didx.at[b]], rb[b], semg[b]).wait()
            _dot_rows(ra[b], rb[b], obuf)
            pltpu.sync_copy(obuf, out_t.at[pl.ds(wid * nch * CHUNK + t * CHUNK,
                                                 CHUNK)])

        fire_idx(0, 0)
        fire_idx(1, 1)
        wait_idx(0, 0)
        fire_g(0)

        def body(p, c):
            k1 = 2 * p + 1
            wait_idx(k1, 1)
            fire_g(1)
            work(k1 - 1, 0)

            @pl.when(p < npair - 1)
            def _a():
                fire_idx(k1 + 1, 0)
                wait_idx(k1 + 1, 0)
                fire_g(0)

            work(k1, 1)

            @pl.when(p < npair - 1)
            def _b():
                fire_idx(k1 + 2, 1)

            return c

        lax.fori_loop(0, npair, body, None)

    return k(ta, src2d, dst2d)


def _rb_spec():
    return pl.BlockSpec((ROW_BLK, D), lambda i: (i, 0))


def _col_spec():
    return pl.BlockSpec((ROW_BLK, 1), lambda i: (i, 0))


def _w_spec():
    return pl.BlockSpec((D, D), lambda i: (0, 0))


def _p_spec():
    return pl.BlockSpec((NCORE, ROW_BLK, D), lambda i: (0, i, 0))


def _tc_stage1(ue, ie, te, au, ai, degp, W_r1, W_rb1, W_t1, W_a1r, W_a1rb,
               W_gat, al, ar):
    def body(ue_r, ie_r, te_r, au_r, ai_r, dp_r, wr1, wrb1, wt1, wa1r, wa1rb,
             wg, al_r, ar_r,
             t1_o, t2_o, t3_o, t4_o, t5_o, feat_o, el_o, er_o,
             frs_o, frd_o, fts_o, ftd_o, fss_o, fsd_o):
        dp = dp_r[...]
        dru = dp[0, 0] + dp[1, 0]
        dri = dp[0, 1] + dp[1, 1]
        dts = dp[0, 2] + dp[1, 2]
        dtd = dp[0, 3] + dp[1, 3]
        frs = lax.rsqrt(jnp.maximum(dru, 1.0))
        frd = lax.rsqrt(jnp.maximum(dri, 1.0))
        fts = lax.rsqrt(jnp.maximum(dts, 1.0))
        ftd = lax.rsqrt(jnp.maximum(dtd, 1.0))
        fss = lax.rsqrt(dts + 1.0)
        fsd = lax.rsqrt(dtd + 1.0)
        frs_o[...] = frs[:, None]
        frd_o[...] = frd[:, None]
        fts_o[...] = fts[:, None]
        ftd_o[...] = ftd[:, None]
        fss_o[...] = fss[:, None]
        fsd_o[...] = fsd[:, None]
        ue_b = ue_r[...]
        ie_b = ie_r[...]
        te_b = te_r[...]
        t1_o[...] = jnp.dot(ue_b, wr1[...], preferred_element_type=jnp.float32) * frs[:, None]
        t2_o[...] = jnp.dot(ie_b, wrb1[...], preferred_element_type=jnp.float32) * frd[:, None]
        t3_o[...] = jnp.dot(ue_b, wt1[...], preferred_element_type=jnp.float32) * fts[:, None]
        t4_o[...] = jnp.dot(au_r[...], wa1r[...], preferred_element_type=jnp.float32) * frs[:, None]
        t5_o[...] = jnp.dot(ai_r[...], wa1rb[...], preferred_element_type=jnp.float32) * frd[:, None]
        feat = jnp.dot(te_b, wg[...], preferred_element_type=jnp.float32)
        feat_o[...] = feat
        el_o[...] = jnp.dot(feat, al_r[...], preferred_element_type=jnp.float32)
        er_o[...] = jnp.dot(feat, ar_r[...], preferred_element_type=jnp.float32)

    rb = jax.ShapeDtypeStruct((N_PAD, D), jnp.float32)
    col = jax.ShapeDtypeStruct((N_PAD, 1), jnp.float32)
    return pl.pallas_call(
        body,
        grid=(GRID,),
        in_specs=[_rb_spec()] * 5
        + [pl.BlockSpec((NCORE, 4, ROW_BLK), lambda i: (0, 0, i))]
        + [_w_spec()] * 6
        + [pl.BlockSpec((D, 1), lambda i: (0, 0))] * 2,
        out_specs=[_rb_spec()] * 6 + [_col_spec()] * 2 + [_col_spec()] * 6,
        out_shape=[rb] * 6 + [col] * 8,
    )(ue, ie, te, au, ai, degp, W_r1, W_rb1, W_t1, W_a1r, W_a1rb, W_gat, al, ar)


def _tc_stage2(P1, P2, P3, P4, P5, Pg, Ps, frs, frd, fts, ftd, fss, fsd,
               W_r2, W_rb2, W_t2, W_a2r, W_a2rb, W_tg2):
    def body(p1, p2, p3, p4, p5, pg, ps, frs_r, frd_r, fts_r, ftd_r, fss_r,
             fsd_r, wr2, wrb2, wt2, wa2r, wa2rb, wtg2,
             t7_o, t8_o, t9_o, t10_o, t11_o, t12_o, xiid1_o, aiid1_o):
        frs_b = frs_r[...]
        frd_b = frd_r[...]
        x_iid1 = _leaky((p1[0] + p1[1]) * frd_b)
        x_uid1 = 0.5 * (_leaky((p2[0] + p2[1]) * frs_b)
                        + _leaky((p3[0] + p3[1]) * ftd_r[...]))
        a_iid1 = _leaky((p4[0] + p4[1]) * frd_b)
        a_uid1 = _leaky((p5[0] + p5[1]) * frs_b)
        s = ps[0] + ps[1]
        gat1 = _leaky((pg[0] + pg[1]) / (s[:, None] + 1e-9))
        t7_o[...] = jnp.dot(x_uid1, wr2[...], preferred_element_type=jnp.float32) * frs_b
        t8_o[...] = jnp.dot(x_iid1, wrb2[...], preferred_element_type=jnp.float32) * frd_b
        t9_o[...] = jnp.dot(x_uid1, wt2[...], preferred_element_type=jnp.float32) * fts_r[...]
        t10_o[...] = jnp.dot(a_uid1, wa2r[...], preferred_element_type=jnp.float32) * frs_b
        t11_o[...] = jnp.dot(a_iid1, wa2rb[...], preferred_element_type=jnp.float32) * frd_b
        t12_o[...] = jnp.dot(gat1, wtg2[...], preferred_element_type=jnp.float32) * fss_r[...]
        xiid1_o[...] = x_iid1
        aiid1_o[...] = a_iid1

    rb = jax.ShapeDtypeStruct((N_PAD, D), jnp.float32)
    return pl.pallas_call(
        body,
        grid=(GRID,),
        in_specs=[_p_spec()] * 6
        + [pl.BlockSpec((NCORE, ROW_BLK), lambda i: (0, i))]
        + [_col_spec()] * 6 + [_w_spec()] * 6,
        out_specs=[_rb_spec()] * 8,
        out_shape=[rb] * 8,
    )(P1, P2, P3, P4, P5, Pg, Ps, frs, frd, fts, ftd, fss, fsd,
      W_r2, W_rb2, W_t2, W_a2r, W_a2rb, W_tg2)


def _tc_stage3(P7, P8, P9, P10, P11, P12, frs, frd, ftd, fsd):
    def body(p7, p8, p9, p10, p11, p12, frs_r, frd_r, ftd_r, fsd_r,
             xu_o, xi_o, au_o, ai_o, t_o):
        frs_b = frs_r[...]
        frd_b = frd_r[...]
        xi_o[...] = _leaky((p7[0] + p7[1]) * frd_b)
        xu_o[...] = jnp.maximum(_leaky((p8[0] + p8[1]) * frs_b),
                                _leaky((p9[0] + p9[1]) * ftd_r[...]))
        ai_o[...] = _leaky((p10[0] + p10[1]) * frd_b)
        au_o[...] = _leaky((p11[0] + p11[1]) * frs_b)
        t_o[...] = _leaky((p12[0] + p12[1]) * fsd_r[...])

    rb = jax.ShapeDtypeStruct((N_PAD, D), jnp.float32)
    return pl.pallas_call(
        body,
        grid=(GRID,),
        in_specs=[_p_spec()] * 6 + [_col_spec()] * 4,
        out_specs=[_rb_spec()] * 5,
        out_shape=[rb] * 5,
    )(P7, P8, P9, P10, P11, P12, frs, frd, ftd, fsd)


def _tc_edge_losses(ratings2d, pos2d, att2d, tr2d):
    rows = ratings2d.shape[0]
    blk = rows

    def body(rt_r, po_r, at_r, tr_r, pos_o, sums_o):
        rt = rt_r[...]
        po = po_r[...]
        err = rt - (po + MEAN_RATE)
        pos_o[...] = po + MEAN_RATE
        at = at_r[...]
        att_s = 1.0 / (1.0 + jnp.exp(-at))
        tgt = 1.0 / (1.0 + jnp.exp(-(rt - MEAN_RATE)))
        tr = tr_r[...]
        sp = jnp.maximum(-tr, 0.0) + jnp.log(1.0 + jnp.exp(-jnp.abs(tr)))
        sg = 1.0 / (1.0 + jnp.exp(-tr))
        upd = jnp.stack([
            jnp.sum(err * err, axis=0),
            jnp.sum(jnp.abs(err), axis=0),
            jnp.sum((att_s - tgt) ** 2, axis=0),
            jnp.sum(sp, axis=0),
            jnp.sum(sg, axis=0),
            jnp.zeros((D,), jnp.float32),
            jnp.zeros((D,), jnp.float32),
            jnp.zeros((D,), jnp.float32),
        ])
        sums_o[...] = upd

    return pl.pallas_call(
        body,
        grid=(1,),
        in_specs=[pl.BlockSpec((blk, D), lambda i: (i, 0))] * 4,
        out_specs=[pl.BlockSpec((blk, D), lambda i: (i, 0)),
                   pl.BlockSpec((8, D), lambda i: (0, 0))],
        out_shape=[jax.ShapeDtypeStruct((rows, D), jnp.float32),
                   jax.ShapeDtypeStruct((8, D), jnp.float32)],
    )(ratings2d, pos2d, att2d, tr2d)


def _tc_table_sums(xu, xi, t, au, ai):
    def body(xu_r, xi_r, t_r, au_r, ai_r, sums_o):
        i = pl.program_id(0)

        @pl.when(i == 0)
        def _():
            sums_o[...] = jnp.zeros((8, D), jnp.float32)

        xi_b = xi_r[...]
        ai_b = ai_r[...]
        reg = (jnp.sum(jnp.abs(xu_r[...]), axis=0) + jnp.sum(jnp.abs(xi_b), axis=0)
               + jnp.sum(jnp.abs(t_r[...]), axis=0) + jnp.sum(jnp.abs(au_r[...]), axis=0)
               + jnp.sum(jnp.abs(ai_b), axis=0))
        ax = jnp.sum(jnp.abs(xi_b - ai_b), axis=0)
        z = jnp.zeros((D,), jnp.float32)
        sums_o[...] = sums_o[...] + jnp.stack([reg, ax, z, z, z, z, z, z])

    return pl.pallas_call(
        body,
        grid=(GRID,),
        in_specs=[_rb_spec()] * 5,
        out_specs=pl.BlockSpec((8, D), lambda i: (0, 0)),
        out_shape=jax.ShapeDtypeStruct((8, D), jnp.float32),
    )(xu, xi, t, au, ai)


def kernel(user_emb, item_emb, trust_emb, a_emb_uid, a_emb_iid, ratings,
           rated_edge_index, trust_edge_index,
           W_r1, W_rb1, W_t1, W_r2, W_rb2, W_t2,
           W_a1r, W_a1rb, W_a2r, W_a2rb, W_gat, W_tg2, attn_l, attn_r):
    f32 = jnp.float32

    def pad_rows(x):
        return jnp.concatenate([x, jnp.zeros((N_PAD - x.shape[0], D), f32)])

    ue = pad_rows(user_emb)
    ie = pad_rows(item_emb)
    te = pad_rows(trust_emb)
    au = pad_rows(a_emb_uid)
    ai = pad_rows(a_emb_iid)

    rs = rated_edge_index[0].astype(jnp.int32)
    rd = rated_edge_index[1].astype(jnp.int32)
    ts = trust_edge_index[0].astype(jnp.int32)
    td = trust_edge_index[1].astype(jnp.int32)

    grain = NW * CHUNK * 4
    e_pad_r = ((E_R + grain - 1) // grain) * grain
    rs_p = _pad_edges(rs, e_pad_r, PAD_SRC).reshape(-1, CHUNK)
    rd_p = _pad_edges(rd, e_pad_r, PAD_DST).reshape(-1, CHUNK)
    ts_p = _pad_edges(ts, e_pad_r, PAD_SRC).reshape(-1, CHUNK)
    td_p = _pad_edges(td, e_pad_r, PAD_DST).reshape(-1, CHUNK)

    sl = jnp.arange(N_U, dtype=jnp.int32)
    tsl_s = jnp.concatenate([ts, sl])
    tsl_d = jnp.concatenate([td, sl])
    e_pad_t = ((tsl_s.shape[0] + grain - 1) // grain) * grain
    tsl_s_p = _pad_edges(tsl_s, e_pad_t, PAD_SRC).reshape(-1, CHUNK)
    tsl_d_p = _pad_edges(tsl_d, e_pad_t, PAD_DST).reshape(-1, CHUNK)

    idx4 = jnp.stack([rs_p, rd_p, ts_p, td_p])
    degp = _sc_degrees(idx4)


    (tb_r1, tb_rb1, tb_t1, tb_a1r, tb_a1rb, feat,
     el2, er2, frs, frd, fts, ftd, fss, fsd) = _tc_stage1(
        ue, ie, te, au, ai, degp, W_r1, W_rb1, W_t1, W_a1r, W_a1rb, W_gat,
        attn_l.reshape(D, 1), attn_r.reshape(D, 1))

    PA = _sc_segsum_multi(
        [tb_r1, tb_rb1, tb_t1, tb_a1r, tb_a1rb],
        [(rs_p, rd_p), (rd_p, rs_p), (ts_p, td_p)],
        [0, 1, 2, 0, 1])
    P1, P2, P3, P4, P5 = (PA[j] for j in range(5))
    Pg, Ps = _sc_gat(feat, el2.reshape(N_PAD), er2.reshape(N_PAD),
                     tsl_s_p, tsl_d_p)

    (tb_r2, tb_rb2, tb_t2, tb_a2r, tb_a2rb, tb_tg2,
     x_iid1, a_iid1) = _tc_stage2(
        P1, P2, P3, P4, P5, Pg, Ps, frs, frd, fts, ftd, fss, fsd,
        W_r2, W_rb2, W_t2, W_a2r, W_a2rb, W_tg2)
    del x_iid1, a_iid1

    PB = _sc_segsum_multi(
        [tb_r2, tb_rb2, tb_t2, tb_a2r, tb_a2rb, tb_tg2],
        [(rs_p, rd_p), (rd_p, rs_p), (ts_p, td_p), (tsl_s_p, tsl_d_p)],
        [0, 1, 2, 0, 1, 3])
    P7, P8, P9, P10, P11, P12 = (PB[j] for j in range(6))

    x_uid, x_iid, a_uid, a_iid, t = _tc_stage3(
        P7, P8, P9, P10, P11, P12, frs, frd, ftd, fsd)

    pos_pre, att_pre = _sc_dots2(x_uid, x_iid, a_uid, a_iid, rs_p, rd_p)
    tr_pre = _sc_dots1(t, ts_p, td_p)

    ratings2d = ratings.reshape(E_R // D, D)
    pos2d = pos_pre[:E_R].reshape(E_R // D, D)
    att2d = att_pre[:E_R].reshape(E_R // D, D)
    tr2d = tr_pre[:E_T].reshape(E_T // D, D)
    pos_out2d, esums = _tc_edge_losses(ratings2d, pos2d, att2d, tr2d)
    tsums = _tc_table_sums(x_uid, x_iid, t, a_uid, a_iid)

    rating_loss = jnp.sum(esums[0]) / E_R
    mae = jnp.sum(esums[1]) / E_R
    l_att = jnp.sum(esums[2]) / E_R
    loss_trust = jnp.sum(esums[3]) / E_T
    trust_auc = jnp.sum(esums[4]) / E_T
    trust_ap = trust_auc
    loss_reg = jnp.sum(tsums[0])
    loss_a_x = jnp.sum(tsums[1])
    pos_score = pos_out2d.reshape(E_R)

    return (rating_loss, mae, loss_reg, pos_score, l_att, loss_a_x,
            trust_auc, trust_ap, loss_trust)


# speedup vs baseline: 1.0995x; 1.0060x over previous
import functools

import jax
import jax.numpy as jnp
from jax import lax
from jax.experimental import pallas as pl
from jax.experimental.pallas import tpu as pltpu
from jax.experimental.pallas import tpu_sc as plsc

N_U = 10000
N_I = 10000
D = 128
E_R = 320000
E_T = 320000
MEAN_RATE = 3.5

N_PAD = 10240
NSUB = 16
NCORE = 2
NW = NCORE * NSUB
RPT = N_PAD // NSUB
CHUNK = 128
PAD_SRC = N_U
PAD_DST = 10200
ROW_BLK = 512
GRID = N_PAD // ROW_BLK
C0_SHARE = 0.72


def _leaky(x):
    return jnp.maximum(x, 0.01 * x)


def _mesh():
    return plsc.VectorSubcoreMesh(core_axis_name="c", subcore_axis_name="s")


_GDN = lax.GatherDimensionNumbers(
    offset_dims=(), collapsed_slice_dims=(0,), start_index_map=(0,))


def _splat_lane(vec16, j):
    idx = jnp.full((16, 1), j, jnp.int32)
    return lax.gather(vec16, idx, _GDN, slice_sizes=(1,),
                      mode=lax.GatherScatterMode.PROMISE_IN_BOUNDS)


def _shuffle(vec16, idx):
    return lax.gather(vec16, idx[:, None], _GDN, slice_sizes=(1,),
                      mode=lax.GatherScatterMode.PROMISE_IN_BOUNDS)


def _hsum16(x):
    lanes = jnp.arange(16, dtype=jnp.int32)
    for off in (8, 4, 2, 1):
        x = x + _shuffle(x, lanes ^ off)
    return x


def _fill_vec(ref, n, val):
    v = jnp.full((16,), val, jnp.float32)

    def zb(i, c):
        ref[pl.ds(i * 16, 16)] = v
        return c

    lax.fori_loop(0, n // 16, zb, None)


def _zero_rows(ref):
    z = jnp.zeros((16,), jnp.float32)

    def zb(r, c):
        for col in range(D // 16):
            ref[r, pl.ds(col * 16, 16)] = z
        return c

    lax.fori_loop(0, CHUNK, zb, None)


def _pad_edges(idx, e_pad, fill):
    return jnp.concatenate(
        [idx.astype(jnp.int32), jnp.full((e_pad - idx.shape[0],), fill, jnp.int32)])


def _sc_degrees(idx4):
    nch = idx4.shape[1] // NW

    @functools.partial(
        pl.kernel, mesh=_mesh(),
        out_type=jax.ShapeDtypeStruct((NCORE, 4, N_PAD), jnp.float32),
        scratch_types=[
            [pltpu.VMEM((nch, CHUNK), jnp.int32) for _ in range(4)],
            pltpu.VMEM((CHUNK,), jnp.float32),
            [pltpu.VMEM_SHARED((N_PAD,), jnp.float32) for _ in range(4)],
            [pltpu.SemaphoreType.DMA for _ in range(4)],
        ],
    )
    def k(idx_hbm, out_hbm, idxs, vbuf, accs, sems):
        cid = lax.axis_index("c")
        sid = lax.axis_index("s")
        wid = cid * NSUB + sid
        _fill_vec(vbuf, CHUNK, 0.0)
        for a in accs:
            for b in range(RPT // CHUNK):
                pltpu.sync_copy(vbuf,
                                a.at[pl.ds(sid * RPT + b * CHUNK, CHUNK)])
        for j in range(4):
            pltpu.sync_copy(idx_hbm.at[j, pl.ds(wid * nch, nch)], idxs[j])
        plsc.subcore_barrier()
        _fill_vec(vbuf, CHUNK, 1.0)

        def body(t, c):
            for j in range(4):
                pltpu.async_copy(vbuf, accs[j].at[idxs[j].at[t]],
                                 sems[j], add=True)
            for j in range(4):
                pltpu.make_async_copy(vbuf, accs[j].at[idxs[j].at[t]],
                                      sems[j]).wait()
            return c

        lax.fori_loop(0, nch, body, None)
        plsc.subcore_barrier()
        for j, a in enumerate(accs):
            for b in range(RPT // CHUNK):
                sl = pl.ds(sid * RPT + b * CHUNK, CHUNK)
                pltpu.sync_copy(a.at[sl], vbuf)
                pltpu.sync_copy(vbuf, out_hbm.at[cid, j, sl])

    return k(idx4)


def _segsum_job(tbl, src, dst, out, jslot, ctx):
    cid, sid, sidx, didx, rows, acc, semg, semi = ctx
    nch_all = src.shape[0] // NSUB
    n0 = max(2, 2 * int(round(nch_all * C0_SHARE / 2)))
    n1 = nch_all - n0
    npair = jnp.where(cid == 0, n0 // 2, n1 // 2)
    base = jnp.where(cid == 0, sid * n0, NSUB * n0 + sid * n1)
    r0 = sid * RPT

    def fire_idx(t, b):
        pltpu.async_copy(src.at[base + t], sidx.at[b], semi[b])
        pltpu.async_copy(dst.at[base + t], didx.at[b], semi[b])

    def wait_idx(t, b):
        pltpu.make_async_copy(src.at[base + t], sidx.at[b],
                              semi[b]).wait()
        pltpu.make_async_copy(dst.at[base + t], didx.at[b],
                              semi[b]).wait()

    def fire_g(b):
        pltpu.async_copy(tbl.at[sidx.at[b]], rows[b], semg[b])

    def wait_g(b):
        pltpu.make_async_copy(tbl.at[sidx.at[b]], rows[b], semg[b]).wait()

    _zero_rows(rows[0])
    for b in range(RPT // CHUNK):
        pltpu.sync_copy(rows[0], acc.at[pl.ds(r0 + b * CHUNK, CHUNK)])
    fire_idx(0, 0)
    fire_idx(1, 1)
    plsc.subcore_barrier()
    wait_idx(0, 0)
    fire_g(0)

    def body(p, c):
        k1 = 2 * p + 1
        wait_idx(k1, 1)
        fire_g(1)
        wait_g(0)
        pltpu.sync_copy(rows[0], acc.at[didx.at[0]], add=True)

        @pl.when(p < npair - 1)
        def _a():
            fire_idx(k1 + 1, 0)
            wait_idx(k1 + 1, 0)
            fire_g(0)

        wait_g(1)
        pltpu.sync_copy(rows[1], acc.at[didx.at[1]], add=True)

        @pl.when(p < npair - 1)
        def _b():
            fire_idx(k1 + 2, 1)

        return c

    lax.fori_loop(0, npair, body, None)
    plsc.subcore_barrier()
    for b in range(RPT // CHUNK):
        sl = pl.ds(r0 + b * CHUNK, CHUNK)
        pltpu.sync_copy(acc.at[sl], rows[0])
        pltpu.sync_copy(rows[0], out.at[jslot, cid, sl])
    plsc.subcore_barrier()


def _sc_segsum_multi(tables, edge_pairs, job_edges):
    njobs = len(tables)

    @functools.partial(
        pl.kernel, mesh=_mesh(),
        out_type=jax.ShapeDtypeStruct((njobs, NCORE, N_PAD, D), jnp.float32),
        scratch_types=[
            pltpu.VMEM((2, CHUNK), jnp.int32),
            pltpu.VMEM((2, CHUNK), jnp.int32),
            [pltpu.VMEM((CHUNK, D), jnp.float32) for _ in range(2)],
            pltpu.VMEM_SHARED((N_PAD, D), jnp.float32),
            [pltpu.SemaphoreType.DMA for _ in range(2)],
            [pltpu.SemaphoreType.DMA for _ in range(2)],
        ],
    )
    def k(*refs):
        tbls = refs[:njobs]
        epairs = refs[njobs:njobs + 2 * len(edge_pairs)]
        out = refs[njobs + 2 * len(edge_pairs)]
        sidx, didx, rows, acc, semg, semi = refs[njobs + 2 * len(edge_pairs) + 1:]
        cid = lax.axis_index("c")
        sid = lax.axis_index("s")
        ctx = (cid, sid, sidx, didx, rows, acc, semg, semi)
        for j in range(njobs):
            e = job_edges[j]
            _segsum_job(tbls[j], epairs[2 * e], epairs[2 * e + 1], out, j, ctx)

    flat_edges = []
    for s, d in edge_pairs:
        flat_edges += [s, d]
    return k(*tables, *flat_edges)


def _sc_gat(feat, el, er, src2d, dst2d):
    nch = src2d.shape[0] // NW
    npair = nch // 2

    @functools.partial(
        pl.kernel, mesh=_mesh(),
        out_type=(jax.ShapeDtypeStruct((NCORE, N_PAD, D), jnp.float32),
                  jax.ShapeDtypeStruct((NCORE, N_PAD), jnp.float32)),
        scratch_types=[
            pltpu.VMEM((2, CHUNK), jnp.int32),
            pltpu.VMEM((2, CHUNK), jnp.int32),
            [pltpu.VMEM((CHUNK,), jnp.float32) for _ in range(2)],
            [pltpu.VMEM((CHUNK,), jnp.float32) for _ in range(2)],
            pltpu.VMEM((CHUNK,), jnp.float32),
            [pltpu.VMEM((CHUNK, D), jnp.float32) for _ in range(2)],
            pltpu.VMEM_SHARED((N_PAD, D), jnp.float32),
            pltpu.VMEM_SHARED((N_PAD,), jnp.float32),
            [pltpu.SemaphoreType.DMA for _ in range(2)],
            [pltpu.SemaphoreType.DMA for _ in range(2)],
            [pltpu.SemaphoreType.DMA for _ in range(2)],
            [pltpu.SemaphoreType.DMA for _ in range(2)],
        ],
    )
    def k(feat_h, el_h, er_h, src, dst, out_rows, out_s,
          sidx, didx, elv, erv, wv, rows, acc, sacc, seml, semr, semf, semi):
        cid = lax.axis_index("c")
        sid = lax.axis_index("s")
        wid = cid * NSUB + sid
        r0 = sid * RPT

        def fire_idx(t, b):
            pltpu.async_copy(src.at[wid * nch + t], sidx.at[b], semi[b])
            pltpu.async_copy(dst.at[wid * nch + t], didx.at[b], semi[b])

        def wait_idx(t, b):
            pltpu.make_async_copy(src.at[wid * nch + t], sidx.at[b],
                                  semi[b]).wait()
            pltpu.make_async_copy(dst.at[wid * nch + t], didx.at[b],
                                  semi[b]).wait()

        def fire_g(b):
            pltpu.async_copy(el_h.at[sidx.at[b]], elv[b], seml[b])
            pltpu.async_copy(er_h.at[didx.at[b]], erv[b], semr[b])
            pltpu.async_copy(feat_h.at[sidx.at[b]], rows[b], semf[b])

        def work(b):
            pltpu.make_async_copy(el_h.at[sidx.at[b]], elv[b], seml[b]).wait()
            pltpu.make_async_copy(er_h.at[didx.at[b]], erv[b], semr[b]).wait()
            for c in range(CHUNK // 16):
                sl = pl.ds(c * 16, 16)
                e = elv[b][sl] + erv[b][sl]
                wv[sl] = jnp.exp(jnp.maximum(e, 0.2 * e))
            pltpu.sync_copy(wv, sacc.at[didx.at[b]], add=True)
            pltpu.make_async_copy(feat_h.at[sidx.at[b]], rows[b],
                                  semf[b]).wait()

            def wgrp(g, c2):
                wg = wv[pl.ds(g * 16, 16)]
                for j in range(16):
                    r = g * 16 + j
                    spl = _splat_lane(wg, j)
                    for col in range(D // 16):
                        sl = pl.ds(col * 16, 16)
                        rows[b][r, sl] = rows[b][r, sl] * spl
                return c2

            lax.fori_loop(0, CHUNK // 16, wgrp, None)
            pltpu.sync_copy(rows[b], acc.at[didx.at[b]], add=True)

        _zero_rows(rows[0])
        _fill_vec(wv, CHUNK, 0.0)
        for b in range(RPT // CHUNK):
            pltpu.sync_copy(rows[0], acc.at[pl.ds(r0 + b * CHUNK, CHUNK)])
            pltpu.sync_copy(wv, sacc.at[pl.ds(r0 + b * CHUNK, CHUNK)])
        fire_idx(0, 0)
        fire_idx(1, 1)
        plsc.subcore_barrier()
        wait_idx(0, 0)
        fire_g(0)

        def body(p, c):
            k1 = 2 * p + 1
            wait_idx(k1, 1)
            fire_g(1)
            work(0)

            @pl.when(p < npair - 1)
            def _a():
                fire_idx(k1 + 1, 0)
                wait_idx(k1 + 1, 0)
                fire_g(0)

            work(1)

            @pl.when(p < npair - 1)
            def _b():
                fire_idx(k1 + 2, 1)

            return c

        lax.fori_loop(0, npair, body, None)
        plsc.subcore_barrier()
        for b in range(RPT // CHUNK):
            sl = pl.ds(r0 + b * CHUNK, CHUNK)
            pltpu.sync_copy(acc.at[sl], rows[0])
            pltpu.sync_copy(rows[0], out_rows.at[cid, sl])
            pltpu.sync_copy(sacc.at[sl], wv)
            pltpu.sync_copy(wv, out_s.at[cid, sl])

    return k(feat, el, er, src2d, dst2d)


def _dot_rows(ra, rb, obuf):
    lanes = jnp.arange(16, dtype=jnp.int32)

    def g_body(g, c):
        res = jnp.zeros((16,), jnp.float32)
        for j in range(16):
            r = g * 16 + j
            accv = jnp.zeros((16,), jnp.float32)
            for col in range(D // 16):
                sl = pl.ds(col * 16, 16)
                accv = accv + ra[r, sl] * rb[r, sl]
            res = jnp.where(lanes == j, _hsum16(accv), res)
        obuf[pl.ds(g * 16, 16)] = res
        return c

    lax.fori_loop(0, CHUNK // 16, g_body, None)


def _sc_dots2(xu, xi, au, ai, src2d, dst2d):
    nch = src2d.shape[0] // NW
    npair = nch // 2
    e_pad = NW * nch * CHUNK

    @functools.partial(
        pl.kernel, mesh=_mesh(),
        out_type=(jax.ShapeDtypeStruct((e_pad,), jnp.float32),
                  jax.ShapeDtypeStruct((e_pad,), jnp.float32)),
        scratch_types=[
            pltpu.VMEM((2, CHUNK), jnp.int32),
            pltpu.VMEM((2, CHUNK), jnp.int32),
            [pltpu.VMEM((CHUNK, D), jnp.float32) for _ in range(2)],
            [pltpu.VMEM((CHUNK, D), jnp.float32) for _ in range(2)],
            pltpu.VMEM((CHUNK, D), jnp.float32),
            pltpu.VMEM((CHUNK, D), jnp.float32),
            pltpu.VMEM((CHUNK,), jnp.float32),
            [pltpu.SemaphoreType.DMA for _ in range(2)],
            pltpu.SemaphoreType.DMA,
            [pltpu.SemaphoreType.DMA for _ in range(2)],
        ],
    )
    def k(xu_h, xi_h, au_h, ai_h, src, dst, out_p, out_a,
          sidx, didx, pa, pb, aa, ab, obuf, semp, sema, semi):
        cid = lax.axis_index("c")
        sid = lax.axis_index("s")
        wid = cid * NSUB + sid

        def fire_idx(t, b):
            pltpu.async_copy(src.at[wid * nch + t], sidx.at[b], semi[b])
            pltpu.async_copy(dst.at[wid * nch + t], didx.at[b], semi[b])

        def wait_idx(t, b):
            pltpu.make_async_copy(src.at[wid * nch + t], sidx.at[b],
                                  semi[b]).wait()
            pltpu.make_async_copy(dst.at[wid * nch + t], didx.at[b],
                                  semi[b]).wait()

        def fire_pos(b):
            pltpu.async_copy(xu_h.at[sidx.at[b]], pa[b], semp[b])
            pltpu.async_copy(xi_h.at[didx.at[b]], pb[b], semp[b])

        def work_pos(t, b):
            pltpu.make_async_copy(xu_h.at[sidx.at[b]], pa[b], semp[b]).wait()
            pltpu.make_async_copy(xi_h.at[didx.at[b]], pb[b], semp[b]).wait()
            _dot_rows(pa[b], pb[b], obuf)
            pltpu.sync_copy(obuf, out_p.at[pl.ds(wid * nch * CHUNK + t * CHUNK,
                                                 CHUNK)])

        def fire_att(b):
            pltpu.async_copy(au_h.at[sidx.at[b]], aa, sema)
            pltpu.async_copy(ai_h.at[didx.at[b]], ab, sema)

        def work_att(t, b):
            pltpu.make_async_copy(au_h.at[sidx.at[b]], aa, sema).wait()
            pltpu.make_async_copy(ai_h.at[didx.at[b]], ab, sema).wait()
            _dot_rows(aa, ab, obuf)
            pltpu.sync_copy(obuf, out_a.at[pl.ds(wid * nch * CHUNK + t * CHUNK,
                                                 CHUNK)])

        fire_idx(0, 0)
        fire_idx(1, 1)
        wait_idx(0, 0)
        fire_pos(0)

        def body(p, c):
            k0 = 2 * p
            k1 = k0 + 1
            wait_idx(k1, 1)
            fire_pos(1)
            fire_att(0)
            work_pos(k0, 0)
            work_att(k0, 0)

            @pl.when(p < npair - 1)
            def _a():
                fire_idx(k1 + 1, 0)
                wait_idx(k1 + 1, 0)
                fire_pos(0)

            fire_att(1)
            work_pos(k1, 1)
            work_att(k1, 1)

            @pl.when(p < npair - 1)
            def _b():
                fire_idx(k1 + 2, 1)

            return c

        lax.fori_loop(0, npair, body, None)

    return k(xu, xi, au, ai, src2d, dst2d)


def _sc_dots1(ta, src2d, dst2d):
    nch = src2d.shape[0] // NW
    npair = nch // 2
    e_pad = NW * nch * CHUNK

    @functools.partial(
        pl.kernel, mesh=_mesh(),
        out_type=jax.ShapeDtypeStruct((e_pad,), jnp.float32),
        scratch_types=[
            pltpu.VMEM((2, CHUNK), jnp.int32),
            pltpu.VMEM((2, CHUNK), jnp.int32),
            [pltpu.VMEM((CHUNK, D), jnp.float32) for _ in range(2)],
            [pltpu.VMEM((CHUNK, D), jnp.float32) for _ in range(2)],
            pltpu.VMEM((CHUNK,), jnp.float32),
            [pltpu.SemaphoreType.DMA for _ in range(2)],
            [pltpu.SemaphoreType.DMA for _ in range(2)],
        ],
    )
    def k(t_h, src, dst, out_t, sidx, didx, ra, rb, obuf, semg, semi):
        cid = lax.axis_index("c")
        sid = lax.axis_index("s")
        wid = cid * NSUB + sid

        def fire_idx(t, b):
            pltpu.async_copy(src.at[wid * nch + t], sidx.at[b], semi[b])
            pltpu.async_copy(dst.at[wid * nch + t], didx.at[b], semi[b])

        def wait_idx(t, b):
            pltpu.make_async_copy(src.at[wid * nch + t], sidx.at[b],
                                  semi[b]).wait()
            pltpu.make_async_copy(dst.at[wid * nch + t], didx.at[b],
                                  semi[b]).wait()

        def fire_g(b):
            pltpu.async_copy(t_h.at[sidx.at[b]], ra[b], semg[b])
            pltpu.async_copy(t_h.at[didx.at[b]], rb[b], semg[b])

        def work(t, b):
            pltpu.make_async_copy(t_h.at[sidx.at[b]], ra[b], semg[b]).wait()
            pltpu.make_async_copy(t_h.at[didx.at[b]], rb[b], semg[b]).wait()
            _dot_rows(ra[b], rb[b], obuf)
            pltpu.sync_copy(obuf, out_t.at[pl.ds(wid * nch * CHUNK + t * CHUNK,
                                                 CHUNK)])

        fire_idx(0, 0)
        fire_idx(1, 1)
        wait_idx(0, 0)
        fire_g(0)

        def body(p, c):
            k1 = 2 * p + 1
            wait_idx(k1, 1)
            fire_g(1)
            work(k1 - 1, 0)

            @pl.when(p < npair - 1)
            def _a():
                fire_idx(k1 + 1, 0)
                wait_idx(k1 + 1, 0)
                fire_g(0)

            work(k1, 1)

            @pl.when(p < npair - 1)
            def _b():
                fire_idx(k1 + 2, 1)

            return c

        lax.fori_loop(0, npair, body, None)

    return k(ta, src2d, dst2d)


def _rb_spec():
    return pl.BlockSpec((ROW_BLK, D), lambda i: (i, 0))


def _col_spec():
    return pl.BlockSpec((ROW_BLK, 1), lambda i: (i, 0))


def _w_spec():
    return pl.BlockSpec((D, D), lambda i: (0, 0))


def _p_spec():
    return pl.BlockSpec((NCORE, ROW_BLK, D), lambda i: (0, i, 0))


def _tc_stage1(ue, ie, te, au, ai, degp, W_r1, W_rb1, W_t1, W_a1r, W_a1rb,
               W_gat, al, ar):
    def body(ue_r, ie_r, te_r, au_r, ai_r, dp_r, wr1, wrb1, wt1, wa1r, wa1rb,
             wg, al_r, ar_r,
             t1_o, t2_o, t3_o, t4_o, t5_o, feat_o, el_o, er_o,
             frs_o, frd_o, fts_o, ftd_o, fss_o, fsd_o):
        dp = dp_r[...]
        dru = dp[0, 0] + dp[1, 0]
        dri = dp[0, 1] + dp[1, 1]
        dts = dp[0, 2] + dp[1, 2]
        dtd = dp[0, 3] + dp[1, 3]
        frs = lax.rsqrt(jnp.maximum(dru, 1.0))
        frd = lax.rsqrt(jnp.maximum(dri, 1.0))
        fts = lax.rsqrt(jnp.maximum(dts, 1.0))
        ftd = lax.rsqrt(jnp.maximum(dtd, 1.0))
        fss = lax.rsqrt(dts + 1.0)
        fsd = lax.rsqrt(dtd + 1.0)
        frs_o[...] = frs[:, None]
        frd_o[...] = frd[:, None]
        fts_o[...] = fts[:, None]
        ftd_o[...] = ftd[:, None]
        fss_o[...] = fss[:, None]
        fsd_o[...] = fsd[:, None]
        ue_b = ue_r[...]
        ie_b = ie_r[...]
        te_b = te_r[...]
        t1_o[...] = jnp.dot(ue_b, wr1[...], preferred_element_type=jnp.float32) * frs[:, None]
        t2_o[...] = jnp.dot(ie_b, wrb1[...], preferred_element_type=jnp.float32) * frd[:, None]
        t3_o[...] = jnp.dot(ue_b, wt1[...], preferred_element_type=jnp.float32) * fts[:, None]
        t4_o[...] = jnp.dot(au_r[...], wa1r[...], preferred_element_type=jnp.float32) * frs[:, None]
        t5_o[...] = jnp.dot(ai_r[...], wa1rb[...], preferred_element_type=jnp.float32) * frd[:, None]
        feat = jnp.dot(te_b, wg[...], preferred_element_type=jnp.float32)
        feat_o[...] = feat
        el_o[...] = jnp.dot(feat, al_r[...], preferred_element_type=jnp.float32)
        er_o[...] = jnp.dot(feat, ar_r[...], preferred_element_type=jnp.float32)

    rb = jax.ShapeDtypeStruct((N_PAD, D), jnp.float32)
    col = jax.ShapeDtypeStruct((N_PAD, 1), jnp.float32)
    return pl.pallas_call(
        body,
        grid=(GRID,),
        in_specs=[_rb_spec()] * 5
        + [pl.BlockSpec((NCORE, 4, ROW_BLK), lambda i: (0, 0, i))]
        + [_w_spec()] * 6
        + [pl.BlockSpec((D, 1), lambda i: (0, 0))] * 2,
        out_specs=[_rb_spec()] * 6 + [_col_spec()] * 2 + [_col_spec()] * 6,
        out_shape=[rb] * 6 + [col] * 8,
    )(ue, ie, te, au, ai, degp, W_r1, W_rb1, W_t1, W_a1r, W_a1rb, W_gat, al, ar)


def _tc_stage2(P1, P2, P3, P4, P5, Pg, Ps, frs, frd, fts, ftd, fss, fsd,
               W_r2, W_rb2, W_t2, W_a2r, W_a2rb, W_tg2):
    def body(p1, p2, p3, p4, p5, pg, ps, frs_r, frd_r, fts_r, ftd_r, fss_r,
             fsd_r, wr2, wrb2, wt2, wa2r, wa2rb, wtg2,
             t7_o, t8_o, t9_o, t10_o, t11_o, t12_o, xiid1_o, aiid1_o):
        frs_b = frs_r[...]
        frd_b = frd_r[...]
        x_iid1 = _leaky((p1[0] + p1[1]) * frd_b)
        x_uid1 = 0.5 * (_leaky((p2[0] + p2[1]) * frs_b)
                        + _leaky((p3[0] + p3[1]) * ftd_r[...]))
        a_iid1 = _leaky((p4[0] + p4[1]) * frd_b)
        a_uid1 = _leaky((p5[0] + p5[1]) * frs_b)
        s = ps[0] + ps[1]
        gat1 = _leaky((pg[0] + pg[1]) / (s[:, None] + 1e-9))
        t7_o[...] = jnp.dot(x_uid1, wr2[...], preferred_element_type=jnp.float32) * frs_b
        t8_o[...] = jnp.dot(x_iid1, wrb2[...], preferred_element_type=jnp.float32) * frd_b
        t9_o[...] = jnp.dot(x_uid1, wt2[...], preferred_element_type=jnp.float32) * fts_r[...]
        t10_o[...] = jnp.dot(a_uid1, wa2r[...], preferred_element_type=jnp.float32) * frs_b
        t11_o[...] = jnp.dot(a_iid1, wa2rb[...], preferred_element_type=jnp.float32) * frd_b
        t12_o[...] = jnp.dot(gat1, wtg2[...], preferred_element_type=jnp.float32) * fss_r[...]
        xiid1_o[...] = x_iid1
        aiid1_o[...] = a_iid1

    rb = jax.ShapeDtypeStruct((N_PAD, D), jnp.float32)
    return pl.pallas_call(
        body,
        grid=(GRID,),
        in_specs=[_p_spec()] * 6
        + [pl.BlockSpec((NCORE, ROW_BLK), lambda i: (0, i))]
        + [_col_spec()] * 6 + [_w_spec()] * 6,
        out_specs=[_rb_spec()] * 8,
        out_shape=[rb] * 8,
    )(P1, P2, P3, P4, P5, Pg, Ps, frs, frd, fts, ftd, fss, fsd,
      W_r2, W_rb2, W_t2, W_a2r, W_a2rb, W_tg2)


def _tc_stage3(P7, P8, P9, P10, P11, P12, frs, frd, ftd, fsd):
    def body(p7, p8, p9, p10, p11, p12, frs_r, frd_r, ftd_r, fsd_r,
             xu_o, xi_o, au_o, ai_o, t_o):
        frs_b = frs_r[...]
        frd_b = frd_r[...]
        xi_o[...] = _leaky((p7[0] + p7[1]) * frd_b)
        xu_o[...] = jnp.maximum(_leaky((p8[0] + p8[1]) * frs_b),
                                _leaky((p9[0] + p9[1]) * ftd_r[...]))
        ai_o[...] = _leaky((p10[0] + p10[1]) * frd_b)
        au_o[...] = _leaky((p11[0] + p11[1]) * frs_b)
        t_o[...] = _leaky((p12[0] + p12[1]) * fsd_r[...])

    rb = jax.ShapeDtypeStruct((N_PAD, D), jnp.float32)
    return pl.pallas_call(
        body,
        grid=(GRID,),
        in_specs=[_p_spec()] * 6 + [_col_spec()] * 4,
        out_specs=[_rb_spec()] * 5,
        out_shape=[rb] * 5,
    )(P7, P8, P9, P10, P11, P12, frs, frd, ftd, fsd)


def _tc_edge_losses(ratings2d, pos2d, att2d, tr2d):
    rows = ratings2d.shape[0]
    blk = rows

    def body(rt_r, po_r, at_r, tr_r, pos_o, sums_o):
        rt = rt_r[...]
        po = po_r[...]
        err = rt - (po + MEAN_RATE)
        pos_o[...] = po + MEAN_RATE
        at = at_r[...]
        att_s = 1.0 / (1.0 + jnp.exp(-at))
        tgt = 1.0 / (1.0 + jnp.exp(-(rt - MEAN_RATE)))
        tr = tr_r[...]
        sp = jnp.maximum(-tr, 0.0) + jnp.log(1.0 + jnp.exp(-jnp.abs(tr)))
        sg = 1.0 / (1.0 + jnp.exp(-tr))
        upd = jnp.stack([
            jnp.sum(err * err, axis=0),
            jnp.sum(jnp.abs(err), axis=0),
            jnp.sum((att_s - tgt) ** 2, axis=0),
            jnp.sum(sp, axis=0),
            jnp.sum(sg, axis=0),
            jnp.zeros((D,), jnp.float32),
            jnp.zeros((D,), jnp.float32),
            jnp.zeros((D,), jnp.float32),
        ])
        sums_o[...] = upd

    return pl.pallas_call(
        body,
        grid=(1,),
        in_specs=[pl.BlockSpec((blk, D), lambda i: (i, 0))] * 4,
        out_specs=[pl.BlockSpec((blk, D), lambda i: (i, 0)),
                   pl.BlockSpec((8, D), lambda i: (0, 0))],
        out_shape=[jax.ShapeDtypeStruct((rows, D), jnp.float32),
                   jax.ShapeDtypeStruct((8, D), jnp.float32)],
    )(ratings2d, pos2d, att2d, tr2d)


def _tc_table_sums(xu, xi, t, au, ai):
    def body(xu_r, xi_r, t_r, au_r, ai_r, sums_o):
        i = pl.program_id(0)

        @pl.when(i == 0)
        def _():
            sums_o[...] = jnp.zeros((8, D), jnp.float32)

        xi_b = xi_r[...]
        ai_b = ai_r[...]
        reg = (jnp.sum(jnp.abs(xu_r[...]), axis=0) + jnp.sum(jnp.abs(xi_b), axis=0)
               + jnp.sum(jnp.abs(t_r[...]), axis=0) + jnp.sum(jnp.abs(au_r[...]), axis=0)
               + jnp.sum(jnp.abs(ai_b), axis=0))
        ax = jnp.sum(jnp.abs(xi_b - ai_b), axis=0)
        z = jnp.zeros((D,), jnp.float32)
        sums_o[...] = sums_o[...] + jnp.stack([reg, ax, z, z, z, z, z, z])

    return pl.pallas_call(
        body,
        grid=(GRID,),
        in_specs=[_rb_spec()] * 5,
        out_specs=pl.BlockSpec((8, D), lambda i: (0, 0)),
        out_shape=jax.ShapeDtypeStruct((8, D), jnp.float32),
    )(xu, xi, t, au, ai)


def kernel(user_emb, item_emb, trust_emb, a_emb_uid, a_emb_iid, ratings,
           rated_edge_index, trust_edge_index,
           W_r1, W_rb1, W_t1, W_r2, W_rb2, W_t2,
           W_a1r, W_a1rb, W_a2r, W_a2rb, W_gat, W_tg2, attn_l, attn_r):
    f32 = jnp.float32

    def pad_rows(x):
        return jnp.concatenate([x, jnp.zeros((N_PAD - x.shape[0], D), f32)])

    ue = pad_rows(user_emb)
    ie = pad_rows(item_emb)
    te = pad_rows(trust_emb)
    au = pad_rows(a_emb_uid)
    ai = pad_rows(a_emb_iid)

    rs = rated_edge_index[0].astype(jnp.int32)
    rd = rated_edge_index[1].astype(jnp.int32)
    ts = trust_edge_index[0].astype(jnp.int32)
    td = trust_edge_index[1].astype(jnp.int32)

    grain = NW * CHUNK * 4
    e_pad_r = ((E_R + grain - 1) // grain) * grain
    rs_p = _pad_edges(rs, e_pad_r, PAD_SRC).reshape(-1, CHUNK)
    rd_p = _pad_edges(rd, e_pad_r, PAD_DST).reshape(-1, CHUNK)
    ts_p = _pad_edges(ts, e_pad_r, PAD_SRC).reshape(-1, CHUNK)
    td_p = _pad_edges(td, e_pad_r, PAD_DST).reshape(-1, CHUNK)

    sl = jnp.arange(N_U, dtype=jnp.int32)
    tsl_s = jnp.concatenate([ts, sl])
    tsl_d = jnp.concatenate([td, sl])
    e_pad_t = ((tsl_s.shape[0] + grain - 1) // grain) * grain
    tsl_s_p = _pad_edges(tsl_s, e_pad_t, PAD_SRC).reshape(-1, CHUNK)
    tsl_d_p = _pad_edges(tsl_d, e_pad_t, PAD_DST).reshape(-1, CHUNK)

    idx4 = jnp.stack([rs_p, rd_p, ts_p, td_p])
    degp = _sc_degrees(idx4)


    (tb_r1, tb_rb1, tb_t1, tb_a1r, tb_a1rb, feat,
     el2, er2, frs, frd, fts, ftd, fss, fsd) = _tc_stage1(
        ue, ie, te, au, ai, degp, W_r1, W_rb1, W_t1, W_a1r, W_a1rb, W_gat,
        attn_l.reshape(D, 1), attn_r.reshape(D, 1))

    PA = _sc_segsum_multi(
        [tb_r1, tb_rb1, tb_t1, tb_a1r, tb_a1rb],
        [(rs_p, rd_p), (rd_p, rs_p), (ts_p, td_p)],
        [0, 1, 2, 0, 1])
    P1, P2, P3, P4, P5 = (PA[j] for j in range(5))
    Pg, Ps = _sc_gat(feat, el2.reshape(N_PAD), er2.reshape(N_PAD),
                     tsl_s_p, tsl_d_p)

    (tb_r2, tb_rb2, tb_t2, tb_a2r, tb_a2rb, tb_tg2,
     x_iid1, a_iid1) = _tc_stage2(
        P1, P2, P3, P4, P5, Pg, Ps, frs, frd, fts, ftd, fss, fsd,
        W_r2, W_rb2, W_t2, W_a2r, W_a2rb, W_tg2)
    del x_iid1, a_iid1

    PB = _sc_segsum_multi(
        [tb_r2, tb_rb2, tb_t2, tb_a2r, tb_a2rb, tb_tg2],
        [(rs_p, rd_p), (rd_p, rs_p), (ts_p, td_p), (tsl_s_p, tsl_d_p)],
        [0, 1, 2, 0, 1, 3])
    P7, P8, P9, P10, P11, P12 = (PB[j] for j in range(6))

    x_uid, x_iid, a_uid, a_iid, t = _tc_stage3(
        P7, P8, P9, P10, P11, P12, frs, frd, ftd, fsd)

    pos_pre, att_pre = _sc_dots2(x_uid, x_iid, a_uid, a_iid, rs_p, rd_p)
    tr_pre = _sc_dots1(t, ts_p, td_p)

    ratings2d = ratings.reshape(E_R // D, D)
    pos2d = pos_pre[:E_R].reshape(E_R // D, D)
    att2d = att_pre[:E_R].reshape(E_R // D, D)
    tr2d = tr_pre[:E_T].reshape(E_T // D, D)
    pos_out2d, esums = _tc_edge_losses(ratings2d, pos2d, att2d, tr2d)
    tsums = _tc_table_sums(x_uid, x_iid, t, a_uid, a_iid)

    rating_loss = jnp.sum(esums[0]) / E_R
    mae = jnp.sum(esums[1]) / E_R
    l_att = jnp.sum(esums[2]) / E_R
    loss_trust = jnp.sum(esums[3]) / E_T
    trust_auc = jnp.sum(esums[4]) / E_T
    trust_ap = trust_auc
    loss_reg = jnp.sum(tsums[0])
    loss_a_x = jnp.sum(tsums[1])
    pos_score = pos_out2d.reshape(E_R)

    return (rating_loss, mae, loss_reg, pos_score, l_att, loss_a_x,
            trust_auc, trust_ap, loss_trust)
